# baseline (speedup 1.0000x reference)
import functools
import jax
import jax.numpy as jnp
from jax import lax
from jax.experimental import pallas as pl
from jax.experimental.pallas import tpu as pltpu
from jax.experimental.pallas import tpu_sc as plsc

_N = 50000
_E = 800000
_DIN = 128
_H = 64
_HH = 32
_DE = 8

_NC = 2
_NS = 16
_CH = 128
_EROWS = 6272
_EPAD = _EROWS * _CH
_RPT1 = _EROWS // _NS
_BLK1 = 56
_NB1 = _RPT1 // _BLK1
_ACC = 50048
_TACC = _ACC // _NS
_ZCH = 136
_DACC = 50176
_TDACC = _DACC // _NS
_DCH = 448
_DUMMY = _N
_RPT2 = _EROWS // (_NC * _NS)
_BLK2 = 28
_NB2 = _RPT2 // _BLK2
_GB1 = 2
_SLAB = 28
_NSLAB = _RPT1 // _SLAB
_SBLK = _SLAB // _GB1
_GB2 = 7
_NGB2 = _RPT2 // _GB2

_NBLK = 2000
_EBLK = 3200

_f32 = jnp.float32
_bf16 = jnp.bfloat16


def _sage_sc_body(src_cat, dst_k1, h_tab, agg_out,
                  isl_s, isl_d, big_a, big_b,
                  gsem_a, gsem_b, ssem_a, ssem_b, acc_sh):
    cid = lax.axis_index("c")
    sid = lax.axis_index("s")

    z16 = jnp.zeros((16,), _f32)

    def _zrow(r, c):
        big_a[r, pl.ds(0, 16)] = z16
        big_a[r, pl.ds(16, 16)] = z16
        return c
    lax.fori_loop(0, _ZCH, _zrow, 0)

    base = sid * _TACC

    def _zacc(k, c):
        pltpu.sync_copy(big_a.at[pl.ds(0, _ZCH)],
                        acc_sh.at[pl.ds(base + k * _ZCH, _ZCH)])
        return c
    lax.fori_loop(0, _TACC // _ZCH, _zacc, 0)

    plsc.subcore_barrier()

    row0 = sid * _RPT1
    bufs = (big_a, big_b)
    gsems = (gsem_a, gsem_b)
    ssems = (ssem_a, ssem_b)

    def _fire(b, p):
        for j in range(_GB1):
            pltpu.async_copy(h_tab.at[isl_s.at[b * _GB1 + j]],
                             bufs[p].at[pl.ds(j * _CH, _CH)], gsems[p])

    def _drain_g(p):
        for j in range(_GB1):
            pltpu.make_async_copy(h_tab.at[isl_s.at[j]],
                                  bufs[p].at[pl.ds(j * _CH, _CH)],
                                  gsems[p]).wait()

    def _fire_s(b, p):
        for j in range(_GB1):
            pltpu.async_copy(bufs[p].at[pl.ds(j * _CH, _CH)],
                             acc_sh.at[isl_d.at[b * _GB1 + j]],
                             ssems[p], add=True)

    def _drain_s(p):
        for j in range(_GB1):
            pltpu.make_async_copy(bufs[p].at[pl.ds(j * _CH, _CH)],
                                  acc_sh.at[isl_d.at[j]],
                                  ssems[p]).wait()

    def _slab(si, c):
        r0 = row0 + si * _SLAB
        pltpu.sync_copy(src_cat.at[cid, pl.ds(r0, _SLAB)], isl_s)
        pltpu.sync_copy(dst_k1.at[pl.ds(r0, _SLAB)], isl_d)

        _fire(0, 0)
        _drain_g(0)
        _fire_s(0, 0)
        _fire(1, 1)

        def _steady(i, cc):
            b = 2 + 2 * i
            _drain_g(1)
            _fire_s(b - 1, 1)
            _drain_s(0)
            _fire(b, 0)
            _drain_g(0)
            _fire_s(b, 0)
            _drain_s(1)
            _fire(b + 1, 1)
            return cc
        lax.fori_loop(0, (_SBLK - 2) // 2, _steady, 0)

        _drain_g(1)
        _fire_s(_SBLK - 1, 1)
        _drain_s(0)
        _drain_s(1)
        return c
    lax.fori_loop(0, _NSLAB, _slab, 0)

    plsc.subcore_barrier()

    def _fl(k, c):
        off = base + k * _ZCH
        pltpu.sync_copy(acc_sh.at[pl.ds(off, _ZCH)], big_a.at[pl.ds(0, _ZCH)])
        pltpu.sync_copy(big_a.at[pl.ds(0, _ZCH)], agg_out.at[cid, pl.ds(off, _ZCH)])
        return c
    lax.fori_loop(0, _TACC // _ZCH, _fl, 0)


def _sage_sc(src_cat, dst_k1, h_tab):
    mesh = plsc.VectorSubcoreMesh(core_axis_name="c", subcore_axis_name="s")
    return pl.kernel(
        _sage_sc_body,
        out_type=jax.ShapeDtypeStruct((_NC, _ACC, _HH), _f32),
        mesh=mesh,
        scratch_types=[
            pltpu.VMEM((_SLAB, _CH), jnp.int32),
            pltpu.VMEM((_SLAB, _CH), jnp.int32),
            pltpu.VMEM((_GB1 * _CH, _HH), _f32),
            pltpu.VMEM((_GB1 * _CH, _HH), _f32),
            pltpu.SemaphoreType.DMA,
            pltpu.SemaphoreType.DMA,
            pltpu.SemaphoreType.DMA,
            pltpu.SemaphoreType.DMA,
            pltpu.VMEM_SHARED((_ACC, _HH), _f32),
        ],
        compiler_params=pltpu.CompilerParams(use_tc_tiling_on_sc=False),
        name="sage_segment_sum_sc",
    )(src_cat, dst_k1, h_tab)


def _deg_sc_body(dst_k1, deg_out, idx_d, ones_v, zdeg, deg_sh):
    cid = lax.axis_index("c")
    sid = lax.axis_index("s")

    z16 = jnp.zeros((16,), _f32)
    for k in range(_DCH // 16):
        zdeg[pl.ds(k * 16, 16)] = z16
    for k in range(8):
        ones_v[pl.ds(k * 16, 16)] = jnp.ones((16,), _f32)

    base = sid * _TDACC

    def _zdg(k, c):
        pltpu.sync_copy(zdeg, deg_sh.at[pl.ds(base + k * _DCH, _DCH)])
        return c
    lax.fori_loop(0, _TDACC // _DCH, _zdg, 0)

    plsc.subcore_barrier()

    wid = sid * _NC + cid
    row0 = wid * _RPT2

    def _blk(b, c):
        r0 = row0 + b * _BLK2
        pltpu.sync_copy(dst_k1.at[pl.ds(r0, _BLK2)], idx_d)

        def _chunk(j, cc):
            pltpu.sync_copy(ones_v, deg_sh.at[idx_d.at[j]], add=True)
            return cc
        lax.fori_loop(0, _BLK2, _chunk, 0)
        return c
    lax.fori_loop(0, _NB2, _blk, 0)

    plsc.subcore_barrier()

    def _fd(k, c):
        off = base + k * _DCH
        pltpu.sync_copy(deg_sh.at[pl.ds(off, _DCH)], zdeg)
        pltpu.sync_copy(zdeg, deg_out.at[cid, pl.ds(off, _DCH)])
        return c
    lax.fori_loop(0, _TDACC // _DCH, _fd, 0)


def _deg_sc(dst_k1):
    mesh = plsc.VectorSubcoreMesh(core_axis_name="c", subcore_axis_name="s")
    return pl.kernel(
        _deg_sc_body,
        out_type=jax.ShapeDtypeStruct((_NC, _DACC), _f32),
        mesh=mesh,
        scratch_types=[
            pltpu.VMEM((_BLK2, _CH), jnp.int32),
            pltpu.VMEM((_CH,), _f32),
            pltpu.VMEM((_DCH,), _f32),
            pltpu.VMEM_SHARED((_DACC,), _f32),
        ],
        compiler_params=pltpu.CompilerParams(use_tc_tiling_on_sc=False),
        name="degree_sc",
    )(dst_k1)


def _egather_sc_body(idx2, h_tab, out2, ix_a, ix_b, big_a, big_b,
                     gsem_a, gsem_b, wsem_a, wsem_b):
    cid = lax.axis_index("c")
    sid = lax.axis_index("s")
    wid = sid * _NC + cid
    row0 = wid * _RPT2
    bufs = (big_a, big_b)
    ixs = (ix_a, ix_b)
    gsems = (gsem_a, gsem_b)
    wsems = (wsem_a, wsem_b)
    nseg = _GB2 * _CH

    for g in range(2):
        def _fire(b, p):
            r0 = row0 + b * _GB2
            pltpu.sync_copy(idx2.at[g, pl.ds(r0, _GB2)], ixs[p])
            for j in range(_GB2):
                pltpu.async_copy(h_tab.at[ixs[p].at[j]],
                                 bufs[p].at[pl.ds(j * _CH, _CH)], gsems[p])

        def _drain_g(p):
            for j in range(_GB2):
                pltpu.make_async_copy(h_tab.at[ixs[p].at[j]],
                                      bufs[p].at[pl.ds(j * _CH, _CH)],
                                      gsems[p]).wait()

        def _fire_w(b, p):
            e0 = (row0 + b * _GB2) * _CH
            pltpu.async_copy(bufs[p], out2.at[g, pl.ds(e0, nseg)], wsems[p])

        def _drain_w(p):
            pltpu.make_async_copy(bufs[p], out2.at[g, pl.ds(0, nseg)],
                                  wsems[p]).wait()

        _fire(0, 0)
        _drain_g(0)
        _fire_w(0, 0)
        _fire(1, 1)

        def _steady(i, c):
            b = 2 + 2 * i
            _drain_g(1)
            _fire_w(b - 1, 1)
            _drain_w(0)
            _fire(b, 0)
            _drain_g(0)
            _fire_w(b, 0)
            _drain_w(1)
            _fire(b + 1, 1)
            return c
        lax.fori_loop(0, (_NGB2 - 2) // 2, _steady, 0)

        _drain_g(1)
        _fire_w(_NGB2 - 1, 1)
        _drain_w(0)
        _drain_w(1)


def _egather_sc(idx2, h_tab):
    mesh = plsc.VectorSubcoreMesh(core_axis_name="c", subcore_axis_name="s")
    return pl.kernel(
        _egather_sc_body,
        out_type=jax.ShapeDtypeStruct((2, _EPAD, _H), _bf16),
        mesh=mesh,
        scratch_types=[
            pltpu.VMEM((_GB2, _CH), jnp.int32),
            pltpu.VMEM((_GB2, _CH), jnp.int32),
            pltpu.VMEM((_GB2 * _CH, _H), _bf16),
            pltpu.VMEM((_GB2 * _CH, _H), _bf16),
            pltpu.SemaphoreType.DMA,
            pltpu.SemaphoreType.DMA,
            pltpu.SemaphoreType.DMA,
            pltpu.SemaphoreType.DMA,
        ],
        compiler_params=pltpu.CompilerParams(use_tc_tiling_on_sc=False),
        name="edge_gather_sc",
    )(idx2, h_tab)


def _dot(a, b):
    return jnp.dot(a, b, preferred_element_type=_f32)


def _enc_body(x_ref, w_ref, b_ref, out_ref):
    h = jnp.maximum(_dot(x_ref[...], w_ref[...]) + b_ref[...], 0.0)
    out_ref[0] = h[:, :_HH]
    out_ref[1] = h[:, _HH:]


def _encoder(x, W, b):
    return pl.pallas_call(
        _enc_body,
        grid=(_N // _NBLK,),
        in_specs=[
            pl.BlockSpec((_NBLK, _DIN), lambda i: (i, 0)),
            pl.BlockSpec((_DIN, _H), lambda i: (0, 0)),
            pl.BlockSpec((1, _H), lambda i: (0, 0)),
        ],
        out_specs=pl.BlockSpec((2, _NBLK, _HH), lambda i: (0, i, 0)),
        out_shape=jax.ShapeDtypeStruct((2, _N, _HH), _f32),
    )(x, W, b)


def _sage_tc_body(h_ref, a_ref, d_ref, ws_ref, wn_ref, b_ref, out_ref):
    h = jnp.concatenate([h_ref[0], h_ref[1]], axis=1)
    agg = jnp.concatenate([a_ref[0], a_ref[1]], axis=1)
    r = 1.0 / jnp.maximum(d_ref[0] + d_ref[1], 1.0)
    h1 = jnp.maximum(
        _dot(h, ws_ref[...]) + _dot(agg * r, wn_ref[...]) + b_ref[...], 0.0)
    out_ref[0] = h1[:, :_HH]
    out_ref[1] = h1[:, _HH:]


def _sage_tc(h_st, agg_st, deg, Ws, Wn, b):
    return pl.pallas_call(
        _sage_tc_body,
        grid=(_N // _NBLK,),
        in_specs=[
            pl.BlockSpec((2, _NBLK, _HH), lambda i: (0, i, 0)),
            pl.BlockSpec((2, _NBLK, _HH), lambda i: (0, i, 0)),
            pl.BlockSpec((2, _NBLK, 1), lambda i: (0, i, 0)),
            pl.BlockSpec((_H, _H), lambda i: (0, 0)),
            pl.BlockSpec((_H, _H), lambda i: (0, 0)),
            pl.BlockSpec((1, _H), lambda i: (0, 0)),
        ],
        out_specs=pl.BlockSpec((2, _NBLK, _HH), lambda i: (0, i, 0)),
        out_shape=jax.ShapeDtypeStruct((2, _N, _HH), _f32),
    )(h_st, agg_st, deg, Ws, Wn, b)


def _node_final_body(h_ref, a_ref, d_ref, pobs_ref, pmask_ref,
                     ws_ref, wn_ref, b_ref,
                     wp1_ref, bp1_ref, wp2_ref, bp2_ref,
                     wpah_ref, wpap_ref, bpa1_ref, wpa2_ref, bpa2_ref,
                     hst_ref, hfull_ref, hbf_ref, pp_ref, pa_ref):
    h = jnp.concatenate([h_ref[0], h_ref[1]], axis=1)
    agg = jnp.concatenate([a_ref[0], a_ref[1]], axis=1)
    r = 1.0 / jnp.maximum(d_ref[0] + d_ref[1], 1.0)
    h2 = jnp.maximum(
        _dot(h, ws_ref[...]) + _dot(agg * r, wn_ref[...]) + b_ref[...], 0.0)
    hst_ref[0] = h2[:, :_HH]
    hst_ref[1] = h2[:, _HH:]
    hfull_ref[...] = h2
    hbf_ref[...] = h2.astype(_bf16)
    ph = jnp.maximum(_dot(h2, wp1_ref[...]) + bp1_ref[...], 0.0)
    pp = _dot(ph, wp2_ref[...]) + bp2_ref[...]
    pp_ref[...] = pp
    pobs = pobs_ref[...]
    pres = jnp.abs(pobs - pp)
    p4 = jnp.concatenate([pobs, pp, pres, pmask_ref[...]], axis=1)
    pa = jnp.maximum(
        _dot(h2, wpah_ref[...]) + _dot(p4, wpap_ref[...]) + bpa1_ref[...], 0.0)
    pa_ref[...] = _dot(pa, wpa2_ref[...]) + bpa2_ref[...]


def _node_final(h_st, agg_st, deg, pobs, pmask, Ws, Wn, b,
                Wp1, bp1, Wp2, bp2, Wpah, Wpap, bpa1, Wpa2, bpa2):
    full = lambda r, c: pl.BlockSpec((r, c), lambda i: (0, 0))
    return pl.pallas_call(
        _node_final_body,
        grid=(_N // _NBLK,),
        in_specs=[
            pl.BlockSpec((2, _NBLK, _HH), lambda i: (0, i, 0)),
            pl.BlockSpec((2, _NBLK, _HH), lambda i: (0, i, 0)),
            pl.BlockSpec((2, _NBLK, 1), lambda i: (0, i, 0)),
            pl.BlockSpec((_NBLK, 1), lambda i: (i, 0)),
            pl.BlockSpec((_NBLK, 1), lambda i: (i, 0)),
            full(_H, _H), full(_H, _H), full(1, _H),
            full(_H, _H), full(1, _H), full(_H, 1), full(1, 1),
            full(_H, _HH), full(4, _HH), full(1, _HH), full(_HH, 1),
            full(1, 1),
        ],
        out_specs=[
            pl.BlockSpec((2, _NBLK, _HH), lambda i: (0, i, 0)),
            pl.BlockSpec((_NBLK, _H), lambda i: (i, 0)),
            pl.BlockSpec((_NBLK, _H), lambda i: (i, 0)),
            pl.BlockSpec((_NBLK, 1), lambda i: (i, 0)),
            pl.BlockSpec((_NBLK, 1), lambda i: (i, 0)),
        ],
        out_shape=[
            jax.ShapeDtypeStruct((2, _N, _HH), _f32),
            jax.ShapeDtypeStruct((_N, _H), _f32),
            jax.ShapeDtypeStruct((_N, _H), _bf16),
            jax.ShapeDtypeStruct((_N, 1), _f32),
            jax.ShapeDtypeStruct((_N, 1), _f32),
        ],
    )(h_st, agg_st, deg, pobs, pmask, Ws, Wn, b,
      Wp1, bp1, Wp2, bp2, Wpah, Wpap, bpa1, Wpa2, bpa2)


def _edge_body(g_ref, ea_ref, iso_ref, fobs_ref, fmask_ref,
               wfs_ref, wfd_ref, wfe_ref, bf1_ref, wf2_ref, bf2_ref,
               wqs_ref, wqd_ref, wqq_ref, bq1_ref, wq2_ref, bq2_ref,
               flow_ref, ql_ref):
    se = g_ref[0]
    de = g_ref[1]
    ef = jnp.where(iso_ref[...] > 0.0, ea_ref[...], 0.0).astype(_bf16)
    fh = jnp.maximum(
        _dot(se, wfs_ref[...].astype(_bf16))
        + _dot(de, wfd_ref[...].astype(_bf16))
        + _dot(ef, wfe_ref[...].astype(_bf16)) + bf1_ref[...], 0.0)
    flow = _dot(fh, wf2_ref[...]) + bf2_ref[...]
    flow_ref[...] = flow
    fobs = fobs_ref[...]
    qres = jnp.abs(fobs - flow)
    q4 = jnp.concatenate([fobs, flow, qres, fmask_ref[...]], axis=1)
    qa = jnp.maximum(
        _dot(se, wqs_ref[...].astype(_bf16))
        + _dot(de, wqd_ref[...].astype(_bf16))
        + _dot(q4, wqq_ref[...]) + bq1_ref[...], 0.0)
    ql_ref[...] = _dot(qa, wq2_ref[...]) + bq2_ref[...]


def _edge_tc(g4, ea, iso, fobs, fmask,
             Wfs, Wfd, Wfe, bf1, Wf2, bf2,
             Wqs, Wqd, Wqq, bq1, Wq2, bq2):
    full = lambda r, c: pl.BlockSpec((r, c), lambda i: (0, 0))
    return pl.pallas_call(
        _edge_body,
        grid=(_E // _EBLK,),
        in_specs=[
            pl.BlockSpec((2, _EBLK, _H), lambda i: (0, i, 0)),
            pl.BlockSpec((_EBLK, _DE), lambda i: (i, 0)),
            pl.BlockSpec((_EBLK, 1), lambda i: (i, 0)),
            pl.BlockSpec((_EBLK, 1), lambda i: (i, 0)),
            pl.BlockSpec((_EBLK, 1), lambda i: (i, 0)),
            full(_H, _H), full(_H, _H), full(_DE, _H), full(1, _H),
            full(_H, 1), full(1, 1),
            full(_H, _HH), full(_H, _HH), full(4, _HH), full(1, _HH),
            full(_HH, 1), full(1, 1),
        ],
        out_specs=[
            pl.BlockSpec((_EBLK, 1), lambda i: (i, 0)),
            pl.BlockSpec((_EBLK, 1), lambda i: (i, 0)),
        ],
        out_shape=[
            jax.ShapeDtypeStruct((_E, 1), _f32),
            jax.ShapeDtypeStruct((_E, 1), _f32),
        ],
    )(g4, ea, iso, fobs, fmask,
      Wfs, Wfd, Wfe, bf1, Wf2, bf2,
      Wqs, Wqd, Wqq, bq1, Wq2, bq2)


def kernel(x, edge_index, edge_attr, is_original_edge, pressure_obs, flow_obs,
           pressure_mask, flow_mask, W_enc, b_enc, Ws1, Wn1, b1, Ws2, Wn2, b2,
           Wp1, bp1, Wp2, bp2, Wf1, bf1, Wf2, bf2, Wpa1, bpa1, Wpa2, bpa2,
           Wqa1, bqa1, Wqa2, bqa2):
    src = edge_index[0]
    dst = edge_index[1]
    osrc = jnp.where(is_original_edge, src, 0)
    odst = jnp.where(is_original_edge, dst, 0)
    pad = _EPAD - _E
    zpad_i = jnp.zeros((pad,), jnp.int32)
    srcp = jnp.concatenate([src, zpad_i])
    dstp = jnp.concatenate([dst, jnp.full((pad,), _DUMMY, jnp.int32)])
    osrcp = jnp.concatenate([osrc, zpad_i])
    odstp = jnp.concatenate([odst, zpad_i])
    src_cat = jnp.stack([srcp, srcp + _N]).reshape(2, _EROWS, _CH)
    dst_k1 = dstp.reshape(_EROWS, _CH)
    idx2 = jnp.stack([osrcp, odstp]).reshape(2, _EROWS, _CH)

    h0_st = _encoder(x, W_enc, b_enc.reshape(1, _H))
    deg2 = _deg_sc(dst_k1).reshape(2, _DACC, 1)
    agg1_p = _sage_sc(src_cat, dst_k1, h0_st.reshape(2 * _N, _HH))
    h1_st = _sage_tc(h0_st, agg1_p, deg2, Ws1, Wn1, b1.reshape(1, _H))
    agg2_p = _sage_sc(src_cat, dst_k1, h1_st.reshape(2 * _N, _HH))
    h2_st, h2, h2bf, pp, palog = _node_final(
        h1_st, agg2_p, deg2,
        pressure_obs.reshape(_N, 1), pressure_mask.reshape(_N, 1),
        Ws2, Wn2, b2.reshape(1, _H),
        Wp1, bp1.reshape(1, _H), Wp2, bp2.reshape(1, 1),
        Wpa1[:_H], Wpa1[_H:], bpa1.reshape(1, _HH), Wpa2,
        bpa2.reshape(1, 1))

    g2 = _egather_sc(idx2, h2bf)

    iso = is_original_edge.astype(_f32).reshape(_E, 1)
    fobs = flow_obs.reshape(_E, 1)
    fmask = flow_mask.reshape(_E, 1)

    flow, qlog = _edge_tc(
        g2, edge_attr, iso, fobs, fmask,
        Wf1[:_H], Wf1[_H:2 * _H], Wf1[2 * _H:], bf1.reshape(1, _H),
        Wf2, bf2.reshape(1, 1),
        Wqa1[:_H], Wqa1[_H:2 * _H], Wqa1[2 * _H:], bqa1.reshape(1, _HH),
        Wqa2, bqa2.reshape(1, 1))

    return (pp[:, 0], flow[:, 0], h2, palog[:, 0], qlog[:, 0])


# speedup vs baseline: 1.0528x; 1.0528x over previous
import functools
import jax
import jax.numpy as jnp
from jax import lax
from jax.experimental import pallas as pl
from jax.experimental.pallas import tpu as pltpu
from jax.experimental.pallas import tpu_sc as plsc

_N = 50000
_E = 800000
_DIN = 128
_H = 64
_HH = 32
_DE = 8

_NC = 2
_NS = 16
_CH = 128
_EROWS = 6272
_EPAD = _EROWS * _CH
_RPT1 = _EROWS // _NS
_BLK1 = 56
_NB1 = _RPT1 // _BLK1
_ACC = 50048
_TACC = _ACC // _NS
_ZCH = 136
_DACC = 50176
_TDACC = _DACC // _NS
_DCH = 448
_DUMMY = _N
_RPT2 = _EROWS // (_NC * _NS)
_BLK2 = 28
_NB2 = _RPT2 // _BLK2
_GB1 = 2
_SLAB = 28
_NSLAB = _RPT1 // _SLAB
_SBLK = _SLAB // _GB1
_GB2 = 7
_NGB2 = _RPT2 // _GB2

_NBLK = 2000
_EBLK = 3200

_f32 = jnp.float32
_bf16 = jnp.bfloat16


def _sage_sc_body(src_cat, dst_k1, h_tab, agg_out,
                  isl_s, isl_d, big_a, big_b,
                  gsem_a, gsem_b, ssem_a, ssem_b, acc_sh):
    cid = lax.axis_index("c")
    sid = lax.axis_index("s")

    z16 = jnp.zeros((16,), _f32)

    def _zrow(r, c):
        big_a[r, pl.ds(0, 16)] = z16
        big_a[r, pl.ds(16, 16)] = z16
        return c
    lax.fori_loop(0, _ZCH, _zrow, 0)

    base = sid * _TACC

    def _zacc(k, c):
        pltpu.sync_copy(big_a.at[pl.ds(0, _ZCH)],
                        acc_sh.at[pl.ds(base + k * _ZCH, _ZCH)])
        return c
    lax.fori_loop(0, _TACC // _ZCH, _zacc, 0)

    plsc.subcore_barrier()

    row0 = sid * _RPT1
    bufs = (big_a, big_b)
    gsems = (gsem_a, gsem_b)
    ssems = (ssem_a, ssem_b)

    def _fire(b, p):
        for j in range(_GB1):
            pltpu.async_copy(h_tab.at[isl_s.at[b * _GB1 + j]],
                             bufs[p].at[pl.ds(j * _CH, _CH)], gsems[p])

    def _drain_g(p):
        for j in range(_GB1):
            pltpu.make_async_copy(h_tab.at[isl_s.at[j]],
                                  bufs[p].at[pl.ds(j * _CH, _CH)],
                                  gsems[p]).wait()

    def _fire_s(b, p):
        for j in range(_GB1):
            pltpu.async_copy(bufs[p].at[pl.ds(j * _CH, _CH)],
                             acc_sh.at[isl_d.at[b * _GB1 + j]],
                             ssems[p], add=True)

    def _drain_s(p):
        for j in range(_GB1):
            pltpu.make_async_copy(bufs[p].at[pl.ds(j * _CH, _CH)],
                                  acc_sh.at[isl_d.at[j]],
                                  ssems[p]).wait()

    def _slab(si, c):
        r0 = row0 + si * _SLAB
        pltpu.sync_copy(src_cat.at[cid, pl.ds(r0, _SLAB)], isl_s)
        pltpu.sync_copy(dst_k1.at[pl.ds(r0, _SLAB)], isl_d)

        _fire(0, 0)
        _drain_g(0)
        _fire_s(0, 0)
        _fire(1, 1)

        def _steady(i, cc):
            b = 2 + 2 * i
            _drain_g(1)
            _fire_s(b - 1, 1)
            _drain_s(0)
            _fire(b, 0)
            _drain_g(0)
            _fire_s(b, 0)
            _drain_s(1)
            _fire(b + 1, 1)
            return cc
        lax.fori_loop(0, (_SBLK - 2) // 2, _steady, 0)

        _drain_g(1)
        _fire_s(_SBLK - 1, 1)
        _drain_s(0)
        _drain_s(1)
        return c
    lax.fori_loop(0, _NSLAB, _slab, 0)

    plsc.subcore_barrier()

    def _fl(k, c):
        off = base + k * _ZCH
        pltpu.sync_copy(acc_sh.at[pl.ds(off, _ZCH)], big_a.at[pl.ds(0, _ZCH)])
        pltpu.sync_copy(big_a.at[pl.ds(0, _ZCH)], agg_out.at[cid, pl.ds(off, _ZCH)])
        return c
    lax.fori_loop(0, _TACC // _ZCH, _fl, 0)


def _sage_sc(src_cat, dst_k1, h_tab):
    mesh = plsc.VectorSubcoreMesh(core_axis_name="c", subcore_axis_name="s")
    return pl.kernel(
        _sage_sc_body,
        out_type=jax.ShapeDtypeStruct((_NC, _ACC, _HH), _f32),
        mesh=mesh,
        scratch_types=[
            pltpu.VMEM((_SLAB, _CH), jnp.int32),
            pltpu.VMEM((_SLAB, _CH), jnp.int32),
            pltpu.VMEM((_GB1 * _CH, _HH), _f32),
            pltpu.VMEM((_GB1 * _CH, _HH), _f32),
            pltpu.SemaphoreType.DMA,
            pltpu.SemaphoreType.DMA,
            pltpu.SemaphoreType.DMA,
            pltpu.SemaphoreType.DMA,
            pltpu.VMEM_SHARED((_ACC, _HH), _f32),
        ],
        compiler_params=pltpu.CompilerParams(use_tc_tiling_on_sc=False),
        name="sage_segment_sum_sc",
    )(src_cat, dst_k1, h_tab)


def _deg_sc_body(dst_k1, deg_out, idx_d, ones_v, zdeg, deg_sh):
    cid = lax.axis_index("c")
    sid = lax.axis_index("s")

    z16 = jnp.zeros((16,), _f32)
    for k in range(_DCH // 16):
        zdeg[pl.ds(k * 16, 16)] = z16
    for k in range(8):
        ones_v[pl.ds(k * 16, 16)] = jnp.ones((16,), _f32)

    base = sid * _TDACC

    def _zdg(k, c):
        pltpu.sync_copy(zdeg, deg_sh.at[pl.ds(base + k * _DCH, _DCH)])
        return c
    lax.fori_loop(0, _TDACC // _DCH, _zdg, 0)

    plsc.subcore_barrier()

    wid = sid * _NC + cid
    row0 = wid * _RPT2

    def _blk(b, c):
        r0 = row0 + b * _BLK2
        pltpu.sync_copy(dst_k1.at[pl.ds(r0, _BLK2)], idx_d)

        def _chunk(j, cc):
            pltpu.sync_copy(ones_v, deg_sh.at[idx_d.at[j]], add=True)
            return cc
        lax.fori_loop(0, _BLK2, _chunk, 0)
        return c
    lax.fori_loop(0, _NB2, _blk, 0)

    plsc.subcore_barrier()

    def _fd(k, c):
        off = base + k * _DCH
        pltpu.sync_copy(deg_sh.at[pl.ds(off, _DCH)], zdeg)
        pltpu.sync_copy(zdeg, deg_out.at[cid, pl.ds(off, _DCH)])
        return c
    lax.fori_loop(0, _TDACC // _DCH, _fd, 0)


def _deg_sc(dst_k1):
    mesh = plsc.VectorSubcoreMesh(core_axis_name="c", subcore_axis_name="s")
    return pl.kernel(
        _deg_sc_body,
        out_type=jax.ShapeDtypeStruct((_NC, _DACC), _f32),
        mesh=mesh,
        scratch_types=[
            pltpu.VMEM((_BLK2, _CH), jnp.int32),
            pltpu.VMEM((_CH,), _f32),
            pltpu.VMEM((_DCH,), _f32),
            pltpu.VMEM_SHARED((_DACC,), _f32),
        ],
        compiler_params=pltpu.CompilerParams(use_tc_tiling_on_sc=False),
        name="degree_sc",
    )(dst_k1)


def _egather_sc_body(idx2, h_tab, out2, ix_a, ix_b, big_a, big_b,
                     gsem_a, gsem_b, wsem_a, wsem_b):
    cid = lax.axis_index("c")
    sid = lax.axis_index("s")
    wid = sid * _NC + cid
    row0 = wid * _RPT2
    bufs = (big_a, big_b)
    ixs = (ix_a, ix_b)
    gsems = (gsem_a, gsem_b)
    wsems = (wsem_a, wsem_b)
    nseg = _GB2 * _CH

    for g in range(2):
        def _fire(b, p):
            r0 = row0 + b * _GB2
            pltpu.sync_copy(idx2.at[g, pl.ds(r0, _GB2)], ixs[p])
            for j in range(_GB2):
                pltpu.async_copy(h_tab.at[ixs[p].at[j]],
                                 bufs[p].at[pl.ds(j * _CH, _CH)], gsems[p])

        def _drain_g(p):
            for j in range(_GB2):
                pltpu.make_async_copy(h_tab.at[ixs[p].at[j]],
                                      bufs[p].at[pl.ds(j * _CH, _CH)],
                                      gsems[p]).wait()

        def _fire_w(b, p):
            e0 = (row0 + b * _GB2) * _CH
            pltpu.async_copy(bufs[p], out2.at[g, pl.ds(e0, nseg)], wsems[p])

        def _drain_w(p):
            pltpu.make_async_copy(bufs[p], out2.at[g, pl.ds(0, nseg)],
                                  wsems[p]).wait()

        _fire(0, 0)
        _drain_g(0)
        _fire_w(0, 0)
        _fire(1, 1)

        def _steady(i, c):
            b = 2 + 2 * i
            _drain_g(1)
            _fire_w(b - 1, 1)
            _drain_w(0)
            _fire(b, 0)
            _drain_g(0)
            _fire_w(b, 0)
            _drain_w(1)
            _fire(b + 1, 1)
            return c
        lax.fori_loop(0, (_NGB2 - 2) // 2, _steady, 0)

        _drain_g(1)
        _fire_w(_NGB2 - 1, 1)
        _drain_w(0)
        _drain_w(1)


def _egather_sc(idx2, h_tab):
    mesh = plsc.VectorSubcoreMesh(core_axis_name="c", subcore_axis_name="s")
    return pl.kernel(
        _egather_sc_body,
        out_type=jax.ShapeDtypeStruct((2, _EPAD, _H), _f32),
        mesh=mesh,
        scratch_types=[
            pltpu.VMEM((_GB2, _CH), jnp.int32),
            pltpu.VMEM((_GB2, _CH), jnp.int32),
            pltpu.VMEM((_GB2 * _CH, _H), _f32),
            pltpu.VMEM((_GB2 * _CH, _H), _f32),
            pltpu.SemaphoreType.DMA,
            pltpu.SemaphoreType.DMA,
            pltpu.SemaphoreType.DMA,
            pltpu.SemaphoreType.DMA,
        ],
        compiler_params=pltpu.CompilerParams(use_tc_tiling_on_sc=False),
        name="edge_gather_sc",
    )(idx2, h_tab)


def _dot(a, b):
    return jnp.dot(a, b, preferred_element_type=_f32)


def _enc_body(x_ref, w_ref, b_ref, out_ref):
    h = jnp.maximum(_dot(x_ref[...], w_ref[...]) + b_ref[...], 0.0)
    out_ref[0] = h[:, :_HH]
    out_ref[1] = h[:, _HH:]


def _encoder(x, W, b):
    return pl.pallas_call(
        _enc_body,
        grid=(_N // _NBLK,),
        in_specs=[
            pl.BlockSpec((_NBLK, _DIN), lambda i: (i, 0)),
            pl.BlockSpec((_DIN, _H), lambda i: (0, 0)),
            pl.BlockSpec((1, _H), lambda i: (0, 0)),
        ],
        out_specs=pl.BlockSpec((2, _NBLK, _HH), lambda i: (0, i, 0)),
        out_shape=jax.ShapeDtypeStruct((2, _N, _HH), _f32),
    )(x, W, b)


def _sage_tc_body(h_ref, a_ref, d_ref, ws_ref, wn_ref, b_ref, out_ref):
    h = jnp.concatenate([h_ref[0], h_ref[1]], axis=1)
    agg = jnp.concatenate([a_ref[0], a_ref[1]], axis=1)
    r = 1.0 / jnp.maximum(d_ref[0] + d_ref[1], 1.0)
    h1 = jnp.maximum(
        _dot(h, ws_ref[...]) + _dot(agg * r, wn_ref[...]) + b_ref[...], 0.0)
    out_ref[0] = h1[:, :_HH]
    out_ref[1] = h1[:, _HH:]


def _sage_tc(h_st, agg_st, deg, Ws, Wn, b):
    return pl.pallas_call(
        _sage_tc_body,
        grid=(_N // _NBLK,),
        in_specs=[
            pl.BlockSpec((2, _NBLK, _HH), lambda i: (0, i, 0)),
            pl.BlockSpec((2, _NBLK, _HH), lambda i: (0, i, 0)),
            pl.BlockSpec((2, _NBLK, 1), lambda i: (0, i, 0)),
            pl.BlockSpec((_H, _H), lambda i: (0, 0)),
            pl.BlockSpec((_H, _H), lambda i: (0, 0)),
            pl.BlockSpec((1, _H), lambda i: (0, 0)),
        ],
        out_specs=pl.BlockSpec((2, _NBLK, _HH), lambda i: (0, i, 0)),
        out_shape=jax.ShapeDtypeStruct((2, _N, _HH), _f32),
    )(h_st, agg_st, deg, Ws, Wn, b)


def _node_final_body(h_ref, a_ref, d_ref, pobs_ref, pmask_ref,
                     ws_ref, wn_ref, b_ref,
                     wp1_ref, bp1_ref, wp2_ref, bp2_ref,
                     wpah_ref, wpap_ref, bpa1_ref, wpa2_ref, bpa2_ref,
                     hst_ref, hfull_ref, pp_ref, pa_ref):
    h = jnp.concatenate([h_ref[0], h_ref[1]], axis=1)
    agg = jnp.concatenate([a_ref[0], a_ref[1]], axis=1)
    r = 1.0 / jnp.maximum(d_ref[0] + d_ref[1], 1.0)
    h2 = jnp.maximum(
        _dot(h, ws_ref[...]) + _dot(agg * r, wn_ref[...]) + b_ref[...], 0.0)
    hst_ref[0] = h2[:, :_HH]
    hst_ref[1] = h2[:, _HH:]
    hfull_ref[...] = h2
    ph = jnp.maximum(_dot(h2, wp1_ref[...]) + bp1_ref[...], 0.0)
    pp = _dot(ph, wp2_ref[...]) + bp2_ref[...]
    pp_ref[...] = pp
    pobs = pobs_ref[...]
    pres = jnp.abs(pobs - pp)
    p4 = jnp.concatenate([pobs, pp, pres, pmask_ref[...]], axis=1)
    pa = jnp.maximum(
        _dot(h2, wpah_ref[...]) + _dot(p4, wpap_ref[...]) + bpa1_ref[...], 0.0)
    pa_ref[...] = _dot(pa, wpa2_ref[...]) + bpa2_ref[...]


def _node_final(h_st, agg_st, deg, pobs, pmask, Ws, Wn, b,
                Wp1, bp1, Wp2, bp2, Wpah, Wpap, bpa1, Wpa2, bpa2):
    full = lambda r, c: pl.BlockSpec((r, c), lambda i: (0, 0))
    return pl.pallas_call(
        _node_final_body,
        grid=(_N // _NBLK,),
        in_specs=[
            pl.BlockSpec((2, _NBLK, _HH), lambda i: (0, i, 0)),
            pl.BlockSpec((2, _NBLK, _HH), lambda i: (0, i, 0)),
            pl.BlockSpec((2, _NBLK, 1), lambda i: (0, i, 0)),
            pl.BlockSpec((_NBLK, 1), lambda i: (i, 0)),
            pl.BlockSpec((_NBLK, 1), lambda i: (i, 0)),
            full(_H, _H), full(_H, _H), full(1, _H),
            full(_H, _H), full(1, _H), full(_H, 1), full(1, 1),
            full(_H, _HH), full(4, _HH), full(1, _HH), full(_HH, 1),
            full(1, 1),
        ],
        out_specs=[
            pl.BlockSpec((2, _NBLK, _HH), lambda i: (0, i, 0)),
            pl.BlockSpec((_NBLK, _H), lambda i: (i, 0)),
            pl.BlockSpec((_NBLK, 1), lambda i: (i, 0)),
            pl.BlockSpec((_NBLK, 1), lambda i: (i, 0)),
        ],
        out_shape=[
            jax.ShapeDtypeStruct((2, _N, _HH), _f32),
            jax.ShapeDtypeStruct((_N, _H), _f32),
            jax.ShapeDtypeStruct((_N, 1), _f32),
            jax.ShapeDtypeStruct((_N, 1), _f32),
        ],
    )(h_st, agg_st, deg, pobs, pmask, Ws, Wn, b,
      Wp1, bp1, Wp2, bp2, Wpah, Wpap, bpa1, Wpa2, bpa2)


def _edge_body(g_ref, ea_ref, iso_ref, fobs_ref, fmask_ref,
               wfs_ref, wfd_ref, wfe_ref, bf1_ref, wf2_ref, bf2_ref,
               wqs_ref, wqd_ref, wqq_ref, bq1_ref, wq2_ref, bq2_ref,
               flow_ref, ql_ref):
    se = g_ref[0].astype(_bf16)
    de = g_ref[1].astype(_bf16)
    ef = jnp.where(iso_ref[...] > 0.0, ea_ref[...], 0.0).astype(_bf16)
    fh = jnp.maximum(
        _dot(se, wfs_ref[...].astype(_bf16))
        + _dot(de, wfd_ref[...].astype(_bf16))
        + _dot(ef, wfe_ref[...].astype(_bf16)) + bf1_ref[...], 0.0)
    flow = _dot(fh, wf2_ref[...]) + bf2_ref[...]
    flow_ref[...] = flow
    fobs = fobs_ref[...]
    qres = jnp.abs(fobs - flow)
    q4 = jnp.concatenate([fobs, flow, qres, fmask_ref[...]], axis=1)
    qa = jnp.maximum(
        _dot(se, wqs_ref[...].astype(_bf16))
        + _dot(de, wqd_ref[...].astype(_bf16))
        + _dot(q4, wqq_ref[...]) + bq1_ref[...], 0.0)
    ql_ref[...] = _dot(qa, wq2_ref[...]) + bq2_ref[...]


def _edge_tc(g4, ea, iso, fobs, fmask,
             Wfs, Wfd, Wfe, bf1, Wf2, bf2,
             Wqs, Wqd, Wqq, bq1, Wq2, bq2):
    full = lambda r, c: pl.BlockSpec((r, c), lambda i: (0, 0))
    return pl.pallas_call(
        _edge_body,
        grid=(_E // _EBLK,),
        in_specs=[
            pl.BlockSpec((2, _EBLK, _H), lambda i: (0, i, 0)),
            pl.BlockSpec((_EBLK, _DE), lambda i: (i, 0)),
            pl.BlockSpec((_EBLK, 1), lambda i: (i, 0)),
            pl.BlockSpec((_EBLK, 1), lambda i: (i, 0)),
            pl.BlockSpec((_EBLK, 1), lambda i: (i, 0)),
            full(_H, _H), full(_H, _H), full(_DE, _H), full(1, _H),
            full(_H, 1), full(1, 1),
            full(_H, _HH), full(_H, _HH), full(4, _HH), full(1, _HH),
            full(_HH, 1), full(1, 1),
        ],
        out_specs=[
            pl.BlockSpec((_EBLK, 1), lambda i: (i, 0)),
            pl.BlockSpec((_EBLK, 1), lambda i: (i, 0)),
        ],
        out_shape=[
            jax.ShapeDtypeStruct((_E, 1), _f32),
            jax.ShapeDtypeStruct((_E, 1), _f32),
        ],
    )(g4, ea, iso, fobs, fmask,
      Wfs, Wfd, Wfe, bf1, Wf2, bf2,
      Wqs, Wqd, Wqq, bq1, Wq2, bq2)


def kernel(x, edge_index, edge_attr, is_original_edge, pressure_obs, flow_obs,
           pressure_mask, flow_mask, W_enc, b_enc, Ws1, Wn1, b1, Ws2, Wn2, b2,
           Wp1, bp1, Wp2, bp2, Wf1, bf1, Wf2, bf2, Wpa1, bpa1, Wpa2, bpa2,
           Wqa1, bqa1, Wqa2, bqa2):
    src = edge_index[0]
    dst = edge_index[1]
    osrc = jnp.where(is_original_edge, src, 0)
    odst = jnp.where(is_original_edge, dst, 0)
    pad = _EPAD - _E
    zpad_i = jnp.zeros((pad,), jnp.int32)
    srcp = jnp.concatenate([src, zpad_i])
    dstp = jnp.concatenate([dst, jnp.full((pad,), _DUMMY, jnp.int32)])
    osrcp = jnp.concatenate([osrc, zpad_i])
    odstp = jnp.concatenate([odst, zpad_i])
    src_cat = jnp.stack([srcp, srcp + _N]).reshape(2, _EROWS, _CH)
    dst_k1 = dstp.reshape(_EROWS, _CH)
    idx2 = jnp.stack([osrcp, odstp]).reshape(2, _EROWS, _CH)

    h0_st = _encoder(x, W_enc, b_enc.reshape(1, _H))
    deg2 = _deg_sc(dst_k1).reshape(2, _DACC, 1)
    agg1_p = _sage_sc(src_cat, dst_k1, h0_st.reshape(2 * _N, _HH))
    h1_st = _sage_tc(h0_st, agg1_p, deg2, Ws1, Wn1, b1.reshape(1, _H))
    agg2_p = _sage_sc(src_cat, dst_k1, h1_st.reshape(2 * _N, _HH))
    h2_st, h2, pp, palog = _node_final(
        h1_st, agg2_p, deg2,
        pressure_obs.reshape(_N, 1), pressure_mask.reshape(_N, 1),
        Ws2, Wn2, b2.reshape(1, _H),
        Wp1, bp1.reshape(1, _H), Wp2, bp2.reshape(1, 1),
        Wpa1[:_H], Wpa1[_H:], bpa1.reshape(1, _HH), Wpa2,
        bpa2.reshape(1, 1))

    g2 = _egather_sc(idx2, h2)

    iso = is_original_edge.astype(_f32).reshape(_E, 1)
    fobs = flow_obs.reshape(_E, 1)
    fmask = flow_mask.reshape(_E, 1)

    flow, qlog = _edge_tc(
        g2, edge_attr, iso, fobs, fmask,
        Wf1[:_H], Wf1[_H:2 * _H], Wf1[2 * _H:], bf1.reshape(1, _H),
        Wf2, bf2.reshape(1, 1),
        Wqa1[:_H], Wqa1[_H:2 * _H], Wqa1[2 * _H:], bqa1.reshape(1, _HH),
        Wqa2, bqa2.reshape(1, 1))

    return (pp[:, 0], flow[:, 0], h2, palog[:, 0], qlog[:, 0])


# speedup vs baseline: 1.0536x; 1.0007x over previous
import functools
import jax
import jax.numpy as jnp
from jax import lax
from jax.experimental import pallas as pl
from jax.experimental.pallas import tpu as pltpu
from jax.experimental.pallas import tpu_sc as plsc

_N = 50000
_E = 800000
_DIN = 128
_H = 64
_HH = 32
_DE = 8

_NC = 2
_NS = 16
_CH = 128
_EROWS = 6272
_EPAD = _EROWS * _CH
_RPT1 = _EROWS // _NS
_BLK1 = 56
_NB1 = _RPT1 // _BLK1
_ACC = 50048
_TACC = _ACC // _NS
_ZCH = 136
_DACC = 50176
_TDACC = _DACC // _NS
_DCH = 448
_DUMMY = _N
_RPT2 = _EROWS // (_NC * _NS)
_BLK2 = 28
_NB2 = _RPT2 // _BLK2
_GB1 = 2
_SLAB = 28
_NSLAB = _RPT1 // _SLAB
_SBLK = _SLAB // _GB1
_GB2 = 7
_NGB2 = _RPT2 // _GB2

_NBLK = 2000
_EBLK = 3200

_f32 = jnp.float32
_bf16 = jnp.bfloat16


def _sage_sc_body(src_cat, dst_k1, h_tab, agg_out,
                  isl_s, isl_d, big_a, big_b,
                  gsem_a, gsem_b, ssem_a, ssem_b, acc_sh):
    cid = lax.axis_index("c")
    sid = lax.axis_index("s")

    z16 = jnp.zeros((16,), _f32)

    def _zrow(r, c):
        big_a[r, pl.ds(0, 16)] = z16
        big_a[r, pl.ds(16, 16)] = z16
        return c
    lax.fori_loop(0, _ZCH, _zrow, 0)

    base = sid * _TACC

    def _zacc(k, c):
        pltpu.sync_copy(big_a.at[pl.ds(0, _ZCH)],
                        acc_sh.at[pl.ds(base + k * _ZCH, _ZCH)])
        return c
    lax.fori_loop(0, _TACC // _ZCH, _zacc, 0)

    plsc.subcore_barrier()

    row0 = sid * _RPT1
    bufs = (big_a, big_b)
    gsems = (gsem_a, gsem_b)
    ssems = (ssem_a, ssem_b)

    def _fire(b, p):
        for j in range(_GB1):
            pltpu.async_copy(h_tab.at[isl_s.at[b * _GB1 + j]],
                             bufs[p].at[pl.ds(j * _CH, _CH)], gsems[p])

    def _drain_g(p):
        for j in range(_GB1):
            pltpu.make_async_copy(h_tab.at[isl_s.at[j]],
                                  bufs[p].at[pl.ds(j * _CH, _CH)],
                                  gsems[p]).wait()

    def _fire_s(b, p):
        for j in range(_GB1):
            pltpu.async_copy(bufs[p].at[pl.ds(j * _CH, _CH)],
                             acc_sh.at[isl_d.at[b * _GB1 + j]],
                             ssems[p], add=True)

    def _drain_s(p):
        for j in range(_GB1):
            pltpu.make_async_copy(bufs[p].at[pl.ds(j * _CH, _CH)],
                                  acc_sh.at[isl_d.at[j]],
                                  ssems[p]).wait()

    def _slab(si, c):
        r0 = row0 + si * _SLAB
        pltpu.sync_copy(src_cat.at[cid, pl.ds(r0, _SLAB)], isl_s)
        pltpu.sync_copy(dst_k1.at[pl.ds(r0, _SLAB)], isl_d)

        _fire(0, 0)
        _drain_g(0)
        _fire_s(0, 0)
        _fire(1, 1)

        def _steady(i, cc):
            b = 2 + 2 * i
            _drain_g(1)
            _fire_s(b - 1, 1)
            _drain_s(0)
            _fire(b, 0)
            _drain_g(0)
            _fire_s(b, 0)
            _drain_s(1)
            _fire(b + 1, 1)
            return cc
        lax.fori_loop(0, (_SBLK - 2) // 2, _steady, 0)

        _drain_g(1)
        _fire_s(_SBLK - 1, 1)
        _drain_s(0)
        _drain_s(1)
        return c
    lax.fori_loop(0, _NSLAB, _slab, 0)

    plsc.subcore_barrier()

    def _fl(k, c):
        off = base + k * _ZCH
        pltpu.sync_copy(acc_sh.at[pl.ds(off, _ZCH)], big_a.at[pl.ds(0, _ZCH)])
        pltpu.sync_copy(big_a.at[pl.ds(0, _ZCH)], agg_out.at[cid, pl.ds(off, _ZCH)])
        return c
    lax.fori_loop(0, _TACC // _ZCH, _fl, 0)


def _sage_sc(src_cat, dst_k1, h_tab):
    mesh = plsc.VectorSubcoreMesh(core_axis_name="c", subcore_axis_name="s")
    return pl.kernel(
        _sage_sc_body,
        out_type=jax.ShapeDtypeStruct((_NC, _ACC, _HH), _f32),
        mesh=mesh,
        scratch_types=[
            pltpu.VMEM((_SLAB, _CH), jnp.int32),
            pltpu.VMEM((_SLAB, _CH), jnp.int32),
            pltpu.VMEM((_GB1 * _CH, _HH), _f32),
            pltpu.VMEM((_GB1 * _CH, _HH), _f32),
            pltpu.SemaphoreType.DMA,
            pltpu.SemaphoreType.DMA,
            pltpu.SemaphoreType.DMA,
            pltpu.SemaphoreType.DMA,
            pltpu.VMEM_SHARED((_ACC, _HH), _f32),
        ],
        compiler_params=pltpu.CompilerParams(use_tc_tiling_on_sc=False),
        name="sage_segment_sum_sc",
    )(src_cat, dst_k1, h_tab)


def _deg_sc_body(dst_k1, deg_out, idx_d, ones_v, zdeg, deg_sh):
    cid = lax.axis_index("c")
    sid = lax.axis_index("s")

    z16 = jnp.zeros((16,), _f32)
    for k in range(_DCH // 16):
        zdeg[pl.ds(k * 16, 16)] = z16
    for k in range(8):
        ones_v[pl.ds(k * 16, 16)] = jnp.ones((16,), _f32)

    base = sid * _TDACC

    def _zdg(k, c):
        pltpu.sync_copy(zdeg, deg_sh.at[pl.ds(base + k * _DCH, _DCH)])
        return c
    lax.fori_loop(0, _TDACC // _DCH, _zdg, 0)

    plsc.subcore_barrier()

    wid = sid * _NC + cid
    row0 = wid * _RPT2

    def _blk(b, c):
        r0 = row0 + b * _BLK2
        pltpu.sync_copy(dst_k1.at[pl.ds(r0, _BLK2)], idx_d)

        def _chunk(j, cc):
            pltpu.sync_copy(ones_v, deg_sh.at[idx_d.at[j]], add=True)
            return cc
        lax.fori_loop(0, _BLK2, _chunk, 0)
        return c
    lax.fori_loop(0, _NB2, _blk, 0)

    plsc.subcore_barrier()

    def _fd(k, c):
        off = base + k * _DCH
        pltpu.sync_copy(deg_sh.at[pl.ds(off, _DCH)], zdeg)
        pltpu.sync_copy(zdeg, deg_out.at[cid, pl.ds(off, _DCH)])
        return c
    lax.fori_loop(0, _TDACC // _DCH, _fd, 0)


def _deg_sc(dst_k1):
    mesh = plsc.VectorSubcoreMesh(core_axis_name="c", subcore_axis_name="s")
    return pl.kernel(
        _deg_sc_body,
        out_type=jax.ShapeDtypeStruct((_NC, _DACC), _f32),
        mesh=mesh,
        scratch_types=[
            pltpu.VMEM((_BLK2, _CH), jnp.int32),
            pltpu.VMEM((_CH,), _f32),
            pltpu.VMEM((_DCH,), _f32),
            pltpu.VMEM_SHARED((_DACC,), _f32),
        ],
        compiler_params=pltpu.CompilerParams(use_tc_tiling_on_sc=False),
        name="degree_sc",
    )(dst_k1)


def _egather_sc_body(idx2, h_tab, out2, ix_a, ix_b, big_a, big_b,
                     gsem_a, gsem_b, wsem_a, wsem_b):
    cid = lax.axis_index("c")
    sid = lax.axis_index("s")
    wid = sid * _NC + cid
    row0 = wid * _RPT2
    bufs = (big_a, big_b)
    ixs = (ix_a, ix_b)
    gsems = (gsem_a, gsem_b)
    wsems = (wsem_a, wsem_b)
    nseg = _GB2 * _CH

    for g in range(2):
        def _fire(b, p):
            r0 = row0 + b * _GB2
            pltpu.sync_copy(idx2.at[g, pl.ds(r0, _GB2)], ixs[p])
            for j in range(_GB2):
                pltpu.async_copy(h_tab.at[ixs[p].at[j]],
                                 bufs[p].at[pl.ds(j * _CH, _CH)], gsems[p])

        def _drain_g(p):
            for j in range(_GB2):
                pltpu.make_async_copy(h_tab.at[ixs[p].at[j]],
                                      bufs[p].at[pl.ds(j * _CH, _CH)],
                                      gsems[p]).wait()

        def _fire_w(b, p):
            e0 = (row0 + b * _GB2) * _CH
            pltpu.async_copy(bufs[p], out2.at[g, pl.ds(e0, nseg)], wsems[p])

        def _drain_w(p):
            pltpu.make_async_copy(bufs[p], out2.at[g, pl.ds(0, nseg)],
                                  wsems[p]).wait()

        _fire(0, 0)
        _drain_g(0)
        _fire_w(0, 0)
        _fire(1, 1)

        def _steady(i, c):
            b = 2 + 2 * i
            _drain_g(1)
            _fire_w(b - 1, 1)
            _drain_w(0)
            _fire(b, 0)
            _drain_g(0)
            _fire_w(b, 0)
            _drain_w(1)
            _fire(b + 1, 1)
            return c
        lax.fori_loop(0, (_NGB2 - 2) // 2, _steady, 0)

        _drain_g(1)
        _fire_w(_NGB2 - 1, 1)
        _drain_w(0)
        _drain_w(1)


def _egather_sc(idx2, h_tab):
    mesh = plsc.VectorSubcoreMesh(core_axis_name="c", subcore_axis_name="s")
    return pl.kernel(
        _egather_sc_body,
        out_type=jax.ShapeDtypeStruct((2, _EPAD, _H), _f32),
        mesh=mesh,
        scratch_types=[
            pltpu.VMEM((_GB2, _CH), jnp.int32),
            pltpu.VMEM((_GB2, _CH), jnp.int32),
            pltpu.VMEM((_GB2 * _CH, _H), _f32),
            pltpu.VMEM((_GB2 * _CH, _H), _f32),
            pltpu.SemaphoreType.DMA,
            pltpu.SemaphoreType.DMA,
            pltpu.SemaphoreType.DMA,
            pltpu.SemaphoreType.DMA,
        ],
        compiler_params=pltpu.CompilerParams(use_tc_tiling_on_sc=False),
        name="edge_gather_sc",
    )(idx2, h_tab)


def _dot(a, b):
    return jnp.dot(a, b, preferred_element_type=_f32)


def _enc_body(x_ref, w_ref, b_ref, out_ref):
    h = jnp.maximum(_dot(x_ref[...], w_ref[...]) + b_ref[...], 0.0)
    out_ref[0] = h[:, :_HH]
    out_ref[1] = h[:, _HH:]


def _encoder(x, W, b):
    return pl.pallas_call(
        _enc_body,
        grid=(_N // _NBLK,),
        in_specs=[
            pl.BlockSpec((_NBLK, _DIN), lambda i: (i, 0)),
            pl.BlockSpec((_DIN, _H), lambda i: (0, 0)),
            pl.BlockSpec((1, _H), lambda i: (0, 0)),
        ],
        out_specs=pl.BlockSpec((2, _NBLK, _HH), lambda i: (0, i, 0)),
        out_shape=jax.ShapeDtypeStruct((2, _N, _HH), _f32),
    )(x, W, b)


def _sage_tc_body(h_ref, a_ref, d_ref, ws_ref, wn_ref, b_ref, out_ref):
    h = jnp.concatenate([h_ref[0], h_ref[1]], axis=1)
    agg = jnp.concatenate([a_ref[0], a_ref[1]], axis=1)
    r = 1.0 / jnp.maximum(d_ref[0] + d_ref[1], 1.0)
    h1 = jnp.maximum(
        _dot(h, ws_ref[...]) + _dot(agg * r, wn_ref[...]) + b_ref[...], 0.0)
    out_ref[0] = h1[:, :_HH]
    out_ref[1] = h1[:, _HH:]


def _sage_tc(h_st, agg_st, deg, Ws, Wn, b):
    return pl.pallas_call(
        _sage_tc_body,
        grid=(_N // _NBLK,),
        in_specs=[
            pl.BlockSpec((2, _NBLK, _HH), lambda i: (0, i, 0)),
            pl.BlockSpec((2, _NBLK, _HH), lambda i: (0, i, 0)),
            pl.BlockSpec((2, _NBLK, 1), lambda i: (0, i, 0)),
            pl.BlockSpec((_H, _H), lambda i: (0, 0)),
            pl.BlockSpec((_H, _H), lambda i: (0, 0)),
            pl.BlockSpec((1, _H), lambda i: (0, 0)),
        ],
        out_specs=pl.BlockSpec((2, _NBLK, _HH), lambda i: (0, i, 0)),
        out_shape=jax.ShapeDtypeStruct((2, _N, _HH), _f32),
    )(h_st, agg_st, deg, Ws, Wn, b)


def _node_final_body(h_ref, a_ref, d_ref, pobs_ref, pmask_ref,
                     ws_ref, wn_ref, b_ref,
                     wp1_ref, bp1_ref, wp2_ref, bp2_ref,
                     wpah_ref, wpap_ref, bpa1_ref, wpa2_ref, bpa2_ref,
                     hst_ref, hfull_ref, pp_ref, pa_ref):
    h = jnp.concatenate([h_ref[0], h_ref[1]], axis=1)
    agg = jnp.concatenate([a_ref[0], a_ref[1]], axis=1)
    r = 1.0 / jnp.maximum(d_ref[0] + d_ref[1], 1.0)
    h2 = jnp.maximum(
        _dot(h, ws_ref[...]) + _dot(agg * r, wn_ref[...]) + b_ref[...], 0.0)
    hst_ref[0] = h2[:, :_HH]
    hst_ref[1] = h2[:, _HH:]
    hfull_ref[...] = h2
    ph = jnp.maximum(_dot(h2, wp1_ref[...]) + bp1_ref[...], 0.0)
    pp = _dot(ph, wp2_ref[...]) + bp2_ref[...]
    pp_ref[...] = pp
    pobs = pobs_ref[...]
    pres = jnp.abs(pobs - pp)
    p4 = jnp.concatenate([pobs, pp, pres, pmask_ref[...]], axis=1)
    pa = jnp.maximum(
        _dot(h2, wpah_ref[...]) + _dot(p4, wpap_ref[...]) + bpa1_ref[...], 0.0)
    pa_ref[...] = _dot(pa, wpa2_ref[...]) + bpa2_ref[...]


def _node_final(h_st, agg_st, deg, pobs, pmask, Ws, Wn, b,
                Wp1, bp1, Wp2, bp2, Wpah, Wpap, bpa1, Wpa2, bpa2):
    full = lambda r, c: pl.BlockSpec((r, c), lambda i: (0, 0))
    return pl.pallas_call(
        _node_final_body,
        grid=(_N // _NBLK,),
        in_specs=[
            pl.BlockSpec((2, _NBLK, _HH), lambda i: (0, i, 0)),
            pl.BlockSpec((2, _NBLK, _HH), lambda i: (0, i, 0)),
            pl.BlockSpec((2, _NBLK, 1), lambda i: (0, i, 0)),
            pl.BlockSpec((_NBLK, 1), lambda i: (i, 0)),
            pl.BlockSpec((_NBLK, 1), lambda i: (i, 0)),
            full(_H, _H), full(_H, _H), full(1, _H),
            full(_H, _H), full(1, _H), full(_H, 1), full(1, 1),
            full(_H, _HH), full(4, _HH), full(1, _HH), full(_HH, 1),
            full(1, 1),
        ],
        out_specs=[
            pl.BlockSpec((2, _NBLK, _HH), lambda i: (0, i, 0)),
            pl.BlockSpec((_NBLK, _H), lambda i: (i, 0)),
            pl.BlockSpec((_NBLK, 1), lambda i: (i, 0)),
            pl.BlockSpec((_NBLK, 1), lambda i: (i, 0)),
        ],
        out_shape=[
            jax.ShapeDtypeStruct((2, _N, _HH), _f32),
            jax.ShapeDtypeStruct((_N, _H), _f32),
            jax.ShapeDtypeStruct((_N, 1), _f32),
            jax.ShapeDtypeStruct((_N, 1), _f32),
        ],
    )(h_st, agg_st, deg, pobs, pmask, Ws, Wn, b,
      Wp1, bp1, Wp2, bp2, Wpah, Wpap, bpa1, Wpa2, bpa2)


def _edge_body(g_ref, ea_ref, iso_ref, fobs_ref, fmask_ref,
               wfs_ref, wfd_ref, wfe_ref, bf1_ref, wf2_ref, bf2_ref,
               wqs_ref, wqd_ref, wqq_ref, bq1_ref, wq2_ref, bq2_ref,
               flow_ref, ql_ref):
    se = g_ref[0]
    de = g_ref[1]
    ef = jnp.where(iso_ref[...] > 0.0, ea_ref[...], 0.0)
    fh = jnp.maximum(
        _dot(se, wfs_ref[...]) + _dot(de, wfd_ref[...])
        + _dot(ef, wfe_ref[...]) + bf1_ref[...], 0.0)
    flow = _dot(fh, wf2_ref[...]) + bf2_ref[...]
    flow_ref[...] = flow
    fobs = fobs_ref[...]
    qres = jnp.abs(fobs - flow)
    q4 = jnp.concatenate([fobs, flow, qres, fmask_ref[...]], axis=1)
    qa = jnp.maximum(
        _dot(se, wqs_ref[...]) + _dot(de, wqd_ref[...])
        + _dot(q4, wqq_ref[...]) + bq1_ref[...], 0.0)
    ql_ref[...] = _dot(qa, wq2_ref[...]) + bq2_ref[...]


def _edge_tc(g4, ea, iso, fobs, fmask,
             Wfs, Wfd, Wfe, bf1, Wf2, bf2,
             Wqs, Wqd, Wqq, bq1, Wq2, bq2):
    full = lambda r, c: pl.BlockSpec((r, c), lambda i: (0, 0))
    return pl.pallas_call(
        _edge_body,
        grid=(_E // _EBLK,),
        in_specs=[
            pl.BlockSpec((2, _EBLK, _H), lambda i: (0, i, 0)),
            pl.BlockSpec((_EBLK, _DE), lambda i: (i, 0)),
            pl.BlockSpec((_EBLK, 1), lambda i: (i, 0)),
            pl.BlockSpec((_EBLK, 1), lambda i: (i, 0)),
            pl.BlockSpec((_EBLK, 1), lambda i: (i, 0)),
            full(_H, _H), full(_H, _H), full(_DE, _H), full(1, _H),
            full(_H, 1), full(1, 1),
            full(_H, _HH), full(_H, _HH), full(4, _HH), full(1, _HH),
            full(_HH, 1), full(1, 1),
        ],
        out_specs=[
            pl.BlockSpec((_EBLK, 1), lambda i: (i, 0)),
            pl.BlockSpec((_EBLK, 1), lambda i: (i, 0)),
        ],
        out_shape=[
            jax.ShapeDtypeStruct((_E, 1), _f32),
            jax.ShapeDtypeStruct((_E, 1), _f32),
        ],
    )(g4, ea, iso, fobs, fmask,
      Wfs, Wfd, Wfe, bf1, Wf2, bf2,
      Wqs, Wqd, Wqq, bq1, Wq2, bq2)


def kernel(x, edge_index, edge_attr, is_original_edge, pressure_obs, flow_obs,
           pressure_mask, flow_mask, W_enc, b_enc, Ws1, Wn1, b1, Ws2, Wn2, b2,
           Wp1, bp1, Wp2, bp2, Wf1, bf1, Wf2, bf2, Wpa1, bpa1, Wpa2, bpa2,
           Wqa1, bqa1, Wqa2, bqa2):
    src = edge_index[0]
    dst = edge_index[1]
    osrc = jnp.where(is_original_edge, src, 0)
    odst = jnp.where(is_original_edge, dst, 0)
    pad = _EPAD - _E
    zpad_i = jnp.zeros((pad,), jnp.int32)
    srcp = jnp.concatenate([src, zpad_i])
    dstp = jnp.concatenate([dst, jnp.full((pad,), _DUMMY, jnp.int32)])
    osrcp = jnp.concatenate([osrc, zpad_i])
    odstp = jnp.concatenate([odst, zpad_i])
    src_cat = jnp.stack([srcp, srcp + _N]).reshape(2, _EROWS, _CH)
    dst_k1 = dstp.reshape(_EROWS, _CH)
    idx2 = jnp.stack([osrcp, odstp]).reshape(2, _EROWS, _CH)

    h0_st = _encoder(x, W_enc, b_enc.reshape(1, _H))
    deg2 = _deg_sc(dst_k1).reshape(2, _DACC, 1)
    agg1_p = _sage_sc(src_cat, dst_k1, h0_st.reshape(2 * _N, _HH))
    h1_st = _sage_tc(h0_st, agg1_p, deg2, Ws1, Wn1, b1.reshape(1, _H))
    agg2_p = _sage_sc(src_cat, dst_k1, h1_st.reshape(2 * _N, _HH))
    h2_st, h2, pp, palog = _node_final(
        h1_st, agg2_p, deg2,
        pressure_obs.reshape(_N, 1), pressure_mask.reshape(_N, 1),
        Ws2, Wn2, b2.reshape(1, _H),
        Wp1, bp1.reshape(1, _H), Wp2, bp2.reshape(1, 1),
        Wpa1[:_H], Wpa1[_H:], bpa1.reshape(1, _HH), Wpa2,
        bpa2.reshape(1, 1))

    g2 = _egather_sc(idx2, h2)

    iso = is_original_edge.astype(_f32).reshape(_E, 1)
    fobs = flow_obs.reshape(_E, 1)
    fmask = flow_mask.reshape(_E, 1)

    flow, qlog = _edge_tc(
        g2, edge_attr, iso, fobs, fmask,
        Wf1[:_H], Wf1[_H:2 * _H], Wf1[2 * _H:], bf1.reshape(1, _H),
        Wf2, bf2.reshape(1, 1),
        Wqa1[:_H], Wqa1[_H:2 * _H], Wqa1[2 * _H:], bqa1.reshape(1, _HH),
        Wqa2, bqa2.reshape(1, 1))

    return (pp[:, 0], flow[:, 0], h2, palog[:, 0], qlog[:, 0])


# speedup vs baseline: 1.0902x; 1.0347x over previous
import functools
import jax
import jax.numpy as jnp
from jax import lax
from jax.experimental import pallas as pl
from jax.experimental.pallas import tpu as pltpu
from jax.experimental.pallas import tpu_sc as plsc

_N = 50000
_E = 800000
_DIN = 128
_H = 64
_HH = 32
_DE = 8

_NC = 2
_NS = 16
_CH = 128
_EROWS = 6272
_EPAD = _EROWS * _CH
_RPT1 = _EROWS // _NS
_BLK1 = 56
_NB1 = _RPT1 // _BLK1
_ACC = 50048
_TACC = _ACC // _NS
_ZCH = 136
_DACC = 50176
_TDACC = _DACC // _NS
_DCH = 448
_DUMMY = _N
_RPT2 = _EROWS // (_NC * _NS)
_BLK2 = 28
_NB2 = _RPT2 // _BLK2
_GB1 = 2
_SLAB = 28
_NSLAB = _RPT1 // _SLAB
_SBLK = _SLAB // _GB1
_GB2 = 7
_NGB2 = _RPT2 // _GB2

_NBLK = 2000
_EBLK = 3136

_f32 = jnp.float32
_bf16 = jnp.bfloat16


def _sage_sc_body(with_deg, src_cat, dst_k1, h_tab, *refs):
    if with_deg:
        (agg_out, deg_out, isl_s, isl_d, big_a, big_b,
         gsem_a, gsem_b, ssem_a, ssem_b, ones_v, zdeg,
         acc_sh, deg_sh) = refs
    else:
        (agg_out, isl_s, isl_d, big_a, big_b,
         gsem_a, gsem_b, ssem_a, ssem_b, acc_sh) = refs
    cid = lax.axis_index("c")
    sid = lax.axis_index("s")

    z16 = jnp.zeros((16,), _f32)

    def _zrow(r, c):
        big_a[r, pl.ds(0, 16)] = z16
        big_a[r, pl.ds(16, 16)] = z16
        return c
    lax.fori_loop(0, _ZCH, _zrow, 0)

    base = sid * _TACC

    def _zacc(k, c):
        pltpu.sync_copy(big_a.at[pl.ds(0, _ZCH)],
                        acc_sh.at[pl.ds(base + k * _ZCH, _ZCH)])
        return c
    lax.fori_loop(0, _TACC // _ZCH, _zacc, 0)

    if with_deg:
        for k in range(8):
            ones_v[pl.ds(k * 16, 16)] = jnp.ones((16,), _f32)
        for k in range(8):
            zdeg[pl.ds(k * 16, 16)] = z16
        zdeg[pl.ds(_ZCH - 16, 16)] = z16

        def _zdg(k, c):
            pltpu.sync_copy(zdeg, deg_sh.at[pl.ds(base + k * _ZCH, _ZCH)])
            return c
        lax.fori_loop(0, _TACC // _ZCH, _zdg, 0)

    plsc.subcore_barrier()

    row0 = sid * _RPT1
    bufs = (big_a, big_b)
    gsems = (gsem_a, gsem_b)
    ssems = (ssem_a, ssem_b)

    def _fire(b, p):
        for j in range(_GB1):
            pltpu.async_copy(h_tab.at[isl_s.at[b * _GB1 + j]],
                             bufs[p].at[pl.ds(j * _CH, _CH)], gsems[p])

    def _drain_g(p):
        for j in range(_GB1):
            pltpu.make_async_copy(h_tab.at[isl_s.at[j]],
                                  bufs[p].at[pl.ds(j * _CH, _CH)],
                                  gsems[p]).wait()

    def _fire_s(b, p):
        for j in range(_GB1):
            pltpu.async_copy(bufs[p].at[pl.ds(j * _CH, _CH)],
                             acc_sh.at[isl_d.at[b * _GB1 + j]],
                             ssems[p], add=True)
        if with_deg:
            @pl.when(cid == 0)
            def _():
                for j in range(_GB1):
                    pltpu.sync_copy(ones_v, deg_sh.at[isl_d.at[b * _GB1 + j]],
                                    add=True)

    def _drain_s(p):
        for j in range(_GB1):
            pltpu.make_async_copy(bufs[p].at[pl.ds(j * _CH, _CH)],
                                  acc_sh.at[isl_d.at[j]],
                                  ssems[p]).wait()

    def _slab(si, c):
        r0 = row0 + si * _SLAB
        pltpu.sync_copy(src_cat.at[cid, pl.ds(r0, _SLAB)], isl_s)
        pltpu.sync_copy(dst_k1.at[pl.ds(r0, _SLAB)], isl_d)

        _fire(0, 0)
        _drain_g(0)
        _fire_s(0, 0)
        _fire(1, 1)

        def _steady(i, cc):
            b = 2 + 2 * i
            _drain_g(1)
            _fire_s(b - 1, 1)
            _drain_s(0)
            _fire(b, 0)
            _drain_g(0)
            _fire_s(b, 0)
            _drain_s(1)
            _fire(b + 1, 1)
            return cc
        lax.fori_loop(0, (_SBLK - 2) // 2, _steady, 0)

        _drain_g(1)
        _fire_s(_SBLK - 1, 1)
        _drain_s(0)
        _drain_s(1)
        return c
    lax.fori_loop(0, _NSLAB, _slab, 0)

    plsc.subcore_barrier()

    def _fl(k, c):
        off = base + k * _ZCH
        pltpu.sync_copy(acc_sh.at[pl.ds(off, _ZCH)], big_a.at[pl.ds(0, _ZCH)])
        pltpu.sync_copy(big_a.at[pl.ds(0, _ZCH)], agg_out.at[cid, pl.ds(off, _ZCH)])
        return c
    lax.fori_loop(0, _TACC // _ZCH, _fl, 0)

    if with_deg:
        @pl.when(cid == 0)
        def _():
            def _fd(k, c):
                off = base + k * _ZCH
                pltpu.sync_copy(deg_sh.at[pl.ds(off, _ZCH)], zdeg)
                pltpu.sync_copy(zdeg, deg_out.at[pl.ds(off, _ZCH)])
                return c
            lax.fori_loop(0, _TACC // _ZCH, _fd, 0)


def _sage_sc(src_cat, dst_k1, h_tab, with_deg):
    mesh = plsc.VectorSubcoreMesh(core_axis_name="c", subcore_axis_name="s")
    out_type = [jax.ShapeDtypeStruct((_NC, _ACC, _HH), _f32)]
    scratch = [
        pltpu.VMEM((_SLAB, _CH), jnp.int32),
        pltpu.VMEM((_SLAB, _CH), jnp.int32),
        pltpu.VMEM((_GB1 * _CH, _HH), _f32),
        pltpu.VMEM((_GB1 * _CH, _HH), _f32),
        pltpu.SemaphoreType.DMA,
        pltpu.SemaphoreType.DMA,
        pltpu.SemaphoreType.DMA,
        pltpu.SemaphoreType.DMA,
    ]
    if with_deg:
        out_type.append(jax.ShapeDtypeStruct((_ACC,), _f32))
        scratch += [pltpu.VMEM((_CH,), _f32), pltpu.VMEM((_ZCH,), _f32)]
    scratch.append(pltpu.VMEM_SHARED((_ACC, _HH), _f32))
    if with_deg:
        scratch.append(pltpu.VMEM_SHARED((_ACC,), _f32))
    return pl.kernel(
        functools.partial(_sage_sc_body, with_deg),
        out_type=out_type,
        mesh=mesh,
        scratch_types=scratch,
        compiler_params=pltpu.CompilerParams(use_tc_tiling_on_sc=False),
        name="sage_segment_sum_deg_sc" if with_deg else "sage_segment_sum_sc",
    )(src_cat, dst_k1, h_tab)


def _egather_sc_body(idx2, h_tab, out2, ix_a, ix_b, big_a, big_b,
                     gsem_a, gsem_b, wsem_a, wsem_b):
    cid = lax.axis_index("c")
    sid = lax.axis_index("s")
    wid = sid * _NC + cid
    row0 = wid * _RPT2
    bufs = (big_a, big_b)
    ixs = (ix_a, ix_b)
    gsems = (gsem_a, gsem_b)
    wsems = (wsem_a, wsem_b)
    nseg = _GB2 * _CH

    for g in range(2):
        def _fire(b, p):
            r0 = row0 + b * _GB2
            pltpu.sync_copy(idx2.at[g, pl.ds(r0, _GB2)], ixs[p])
            for j in range(_GB2):
                pltpu.async_copy(h_tab.at[ixs[p].at[j]],
                                 bufs[p].at[pl.ds(j * _CH, _CH)], gsems[p])

        def _drain_g(p):
            for j in range(_GB2):
                pltpu.make_async_copy(h_tab.at[ixs[p].at[j]],
                                      bufs[p].at[pl.ds(j * _CH, _CH)],
                                      gsems[p]).wait()

        def _fire_w(b, p):
            e0 = (row0 + b * _GB2) * _CH
            pltpu.async_copy(bufs[p], out2.at[g, pl.ds(e0, nseg)], wsems[p])

        def _drain_w(p):
            pltpu.make_async_copy(bufs[p], out2.at[g, pl.ds(0, nseg)],
                                  wsems[p]).wait()

        _fire(0, 0)
        _drain_g(0)
        _fire_w(0, 0)
        _fire(1, 1)

        def _steady(i, c):
            b = 2 + 2 * i
            _drain_g(1)
            _fire_w(b - 1, 1)
            _drain_w(0)
            _fire(b, 0)
            _drain_g(0)
            _fire_w(b, 0)
            _drain_w(1)
            _fire(b + 1, 1)
            return c
        lax.fori_loop(0, (_NGB2 - 2) // 2, _steady, 0)

        _drain_g(1)
        _fire_w(_NGB2 - 1, 1)
        _drain_w(0)
        _drain_w(1)


def _egather_sc(idx2, h_tab):
    mesh = plsc.VectorSubcoreMesh(core_axis_name="c", subcore_axis_name="s")
    return pl.kernel(
        _egather_sc_body,
        out_type=jax.ShapeDtypeStruct((2, _EPAD, _H), _f32),
        mesh=mesh,
        scratch_types=[
            pltpu.VMEM((_GB2, _CH), jnp.int32),
            pltpu.VMEM((_GB2, _CH), jnp.int32),
            pltpu.VMEM((_GB2 * _CH, _H), _f32),
            pltpu.VMEM((_GB2 * _CH, _H), _f32),
            pltpu.SemaphoreType.DMA,
            pltpu.SemaphoreType.DMA,
            pltpu.SemaphoreType.DMA,
            pltpu.SemaphoreType.DMA,
        ],
        compiler_params=pltpu.CompilerParams(use_tc_tiling_on_sc=False),
        name="edge_gather_sc",
    )(idx2, h_tab)


def _dot(a, b):
    return jnp.dot(a, b, preferred_element_type=_f32)


def _enc_body(x_ref, w_ref, b_ref, out_ref):
    h = jnp.maximum(_dot(x_ref[...], w_ref[...]) + b_ref[...], 0.0)
    out_ref[0] = h[:, :_HH]
    out_ref[1] = h[:, _HH:]


def _encoder(x, W, b):
    return pl.pallas_call(
        _enc_body,
        grid=(_N // _NBLK,),
        in_specs=[
            pl.BlockSpec((_NBLK, _DIN), lambda i: (i, 0)),
            pl.BlockSpec((_DIN, _H), lambda i: (0, 0)),
            pl.BlockSpec((1, _H), lambda i: (0, 0)),
        ],
        out_specs=pl.BlockSpec((2, _NBLK, _HH), lambda i: (0, i, 0)),
        out_shape=jax.ShapeDtypeStruct((2, _N, _HH), _f32),
    )(x, W, b)


def _sage_tc_body(h_ref, a_ref, d_ref, ws_ref, wn_ref, b_ref, out_ref):
    h = jnp.concatenate([h_ref[0], h_ref[1]], axis=1)
    agg = jnp.concatenate([a_ref[0], a_ref[1]], axis=1)
    r = 1.0 / jnp.maximum(d_ref[...], 1.0)
    h1 = jnp.maximum(
        _dot(h, ws_ref[...]) + _dot(agg * r, wn_ref[...]) + b_ref[...], 0.0)
    out_ref[0] = h1[:, :_HH]
    out_ref[1] = h1[:, _HH:]


def _sage_tc(h_st, agg_st, deg, Ws, Wn, b):
    return pl.pallas_call(
        _sage_tc_body,
        grid=(_N // _NBLK,),
        in_specs=[
            pl.BlockSpec((2, _NBLK, _HH), lambda i: (0, i, 0)),
            pl.BlockSpec((2, _NBLK, _HH), lambda i: (0, i, 0)),
            pl.BlockSpec((_NBLK, 1), lambda i: (i, 0)),
            pl.BlockSpec((_H, _H), lambda i: (0, 0)),
            pl.BlockSpec((_H, _H), lambda i: (0, 0)),
            pl.BlockSpec((1, _H), lambda i: (0, 0)),
        ],
        out_specs=pl.BlockSpec((2, _NBLK, _HH), lambda i: (0, i, 0)),
        out_shape=jax.ShapeDtypeStruct((2, _N, _HH), _f32),
    )(h_st, agg_st, deg, Ws, Wn, b)


def _node_final_body(h_ref, a_ref, d_ref, pobs_ref, pmask_ref,
                     ws_ref, wn_ref, b_ref,
                     wp1_ref, bp1_ref, wp2_ref, bp2_ref,
                     wpah_ref, wpap_ref, bpa1_ref, wpa2_ref, bpa2_ref,
                     hst_ref, hfull_ref, pp_ref, pa_ref):
    h = jnp.concatenate([h_ref[0], h_ref[1]], axis=1)
    agg = jnp.concatenate([a_ref[0], a_ref[1]], axis=1)
    r = 1.0 / jnp.maximum(d_ref[...], 1.0)
    h2 = jnp.maximum(
        _dot(h, ws_ref[...]) + _dot(agg * r, wn_ref[...]) + b_ref[...], 0.0)
    hst_ref[0] = h2[:, :_HH]
    hst_ref[1] = h2[:, _HH:]
    hfull_ref[...] = h2
    ph = jnp.maximum(_dot(h2, wp1_ref[...]) + bp1_ref[...], 0.0)
    pp = _dot(ph, wp2_ref[...]) + bp2_ref[...]
    pp_ref[...] = pp
    pobs = pobs_ref[...]
    pres = jnp.abs(pobs - pp)
    p4 = jnp.concatenate([pobs, pp, pres, pmask_ref[...]], axis=1)
    pa = jnp.maximum(
        _dot(h2, wpah_ref[...]) + _dot(p4, wpap_ref[...]) + bpa1_ref[...], 0.0)
    pa_ref[...] = _dot(pa, wpa2_ref[...]) + bpa2_ref[...]


def _node_final(h_st, agg_st, deg, pobs, pmask, Ws, Wn, b,
                Wp1, bp1, Wp2, bp2, Wpah, Wpap, bpa1, Wpa2, bpa2):
    full = lambda r, c: pl.BlockSpec((r, c), lambda i: (0, 0))
    return pl.pallas_call(
        _node_final_body,
        grid=(_N // _NBLK,),
        in_specs=[
            pl.BlockSpec((2, _NBLK, _HH), lambda i: (0, i, 0)),
            pl.BlockSpec((2, _NBLK, _HH), lambda i: (0, i, 0)),
            pl.BlockSpec((_NBLK, 1), lambda i: (i, 0)),
            pl.BlockSpec((_NBLK, 1), lambda i: (i, 0)),
            pl.BlockSpec((_NBLK, 1), lambda i: (i, 0)),
            full(_H, _H), full(_H, _H), full(1, _H),
            full(_H, _H), full(1, _H), full(_H, 1), full(1, 1),
            full(_H, _HH), full(4, _HH), full(1, _HH), full(_HH, 1),
            full(1, 1),
        ],
        out_specs=[
            pl.BlockSpec((2, _NBLK, _HH), lambda i: (0, i, 0)),
            pl.BlockSpec((_NBLK, _H), lambda i: (i, 0)),
            pl.BlockSpec((_NBLK, 1), lambda i: (i, 0)),
            pl.BlockSpec((_NBLK, 1), lambda i: (i, 0)),
        ],
        out_shape=[
            jax.ShapeDtypeStruct((2, _N, _HH), _f32),
            jax.ShapeDtypeStruct((_N, _H), _f32),
            jax.ShapeDtypeStruct((_N, 1), _f32),
            jax.ShapeDtypeStruct((_N, 1), _f32),
        ],
    )(h_st, agg_st, deg, pobs, pmask, Ws, Wn, b,
      Wp1, bp1, Wp2, bp2, Wpah, Wpap, bpa1, Wpa2, bpa2)


def _edge_body(g_ref, ea_ref, iso_ref, fobs_ref, fmask_ref,
               wfs_ref, wfd_ref, wfe_ref, bf1_ref, wf2_ref, bf2_ref,
               wqs_ref, wqd_ref, wqq_ref, bq1_ref, wq2_ref, bq2_ref,
               flow_ref, ql_ref):
    se = g_ref[0]
    de = g_ref[1]
    ef = jnp.where(iso_ref[...] > 0.0, ea_ref[...], 0.0)
    fh = jnp.maximum(
        _dot(se, wfs_ref[...]) + _dot(de, wfd_ref[...])
        + _dot(ef, wfe_ref[...]) + bf1_ref[...], 0.0)
    flow = _dot(fh, wf2_ref[...]) + bf2_ref[...]
    flow_ref[...] = flow
    fobs = fobs_ref[...]
    qres = jnp.abs(fobs - flow)
    q4 = jnp.concatenate([fobs, flow, qres, fmask_ref[...]], axis=1)
    qa = jnp.maximum(
        _dot(se, wqs_ref[...]) + _dot(de, wqd_ref[...])
        + _dot(q4, wqq_ref[...]) + bq1_ref[...], 0.0)
    ql_ref[...] = _dot(qa, wq2_ref[...]) + bq2_ref[...]


def _edge_tc(g4, ea, iso, fobs, fmask,
             Wfs, Wfd, Wfe, bf1, Wf2, bf2,
             Wqs, Wqd, Wqq, bq1, Wq2, bq2):
    full = lambda r, c: pl.BlockSpec((r, c), lambda i: (0, 0))
    return pl.pallas_call(
        _edge_body,
        grid=(_EPAD // _EBLK,),
        in_specs=[
            pl.BlockSpec((2, _EBLK, _H), lambda i: (0, i, 0)),
            pl.BlockSpec((_EBLK, _DE), lambda i: (i, 0)),
            pl.BlockSpec((_EBLK, 1), lambda i: (i, 0)),
            pl.BlockSpec((_EBLK, 1), lambda i: (i, 0)),
            pl.BlockSpec((_EBLK, 1), lambda i: (i, 0)),
            full(_H, _H), full(_H, _H), full(_DE, _H), full(1, _H),
            full(_H, 1), full(1, 1),
            full(_H, _HH), full(_H, _HH), full(4, _HH), full(1, _HH),
            full(_HH, 1), full(1, 1),
        ],
        out_specs=[
            pl.BlockSpec((_EBLK, 1), lambda i: (i, 0)),
            pl.BlockSpec((_EBLK, 1), lambda i: (i, 0)),
        ],
        out_shape=[
            jax.ShapeDtypeStruct((_EPAD, 1), _f32),
            jax.ShapeDtypeStruct((_EPAD, 1), _f32),
        ],
    )(g4, ea, iso, fobs, fmask,
      Wfs, Wfd, Wfe, bf1, Wf2, bf2,
      Wqs, Wqd, Wqq, bq1, Wq2, bq2)


def kernel(x, edge_index, edge_attr, is_original_edge, pressure_obs, flow_obs,
           pressure_mask, flow_mask, W_enc, b_enc, Ws1, Wn1, b1, Ws2, Wn2, b2,
           Wp1, bp1, Wp2, bp2, Wf1, bf1, Wf2, bf2, Wpa1, bpa1, Wpa2, bpa2,
           Wqa1, bqa1, Wqa2, bqa2):
    src = edge_index[0]
    dst = edge_index[1]
    osrc = jnp.where(is_original_edge, src, 0)
    odst = jnp.where(is_original_edge, dst, 0)
    pad = _EPAD - _E
    zpad_i = jnp.zeros((pad,), jnp.int32)
    srcp = jnp.concatenate([src, zpad_i])
    dstp = jnp.concatenate([dst, jnp.full((pad,), _DUMMY, jnp.int32)])
    osrcp = jnp.concatenate([osrc, zpad_i])
    odstp = jnp.concatenate([odst, zpad_i])
    src_cat = jnp.stack([srcp, srcp + _N]).reshape(2, _EROWS, _CH)
    dst_k1 = dstp.reshape(_EROWS, _CH)
    idx2 = jnp.stack([osrcp, odstp]).reshape(2, _EROWS, _CH)

    h0_st = _encoder(x, W_enc, b_enc.reshape(1, _H))
    agg1_p, deg_p = _sage_sc(src_cat, dst_k1, h0_st.reshape(2 * _N, _HH), True)
    deg = deg_p.reshape(_ACC, 1)
    h1_st = _sage_tc(h0_st, agg1_p, deg, Ws1, Wn1, b1.reshape(1, _H))
    agg2_p, = _sage_sc(src_cat, dst_k1, h1_st.reshape(2 * _N, _HH), False)
    h2_st, h2, pp, palog = _node_final(
        h1_st, agg2_p, deg,
        pressure_obs.reshape(_N, 1), pressure_mask.reshape(_N, 1),
        Ws2, Wn2, b2.reshape(1, _H),
        Wp1, bp1.reshape(1, _H), Wp2, bp2.reshape(1, 1),
        Wpa1[:_H], Wpa1[_H:], bpa1.reshape(1, _HH), Wpa2,
        bpa2.reshape(1, 1))

    g2 = _egather_sc(idx2, h2)

    zpad_f = jnp.zeros((pad,), _f32)
    ea_p = jnp.concatenate([edge_attr, jnp.zeros((pad, _DE), _f32)])
    iso_p = jnp.concatenate([is_original_edge.astype(_f32), zpad_f]
                            ).reshape(_EPAD, 1)
    fobs_p = jnp.concatenate([flow_obs, zpad_f]).reshape(_EPAD, 1)
    fmask_p = jnp.concatenate([flow_mask, zpad_f]).reshape(_EPAD, 1)

    flow, qlog = _edge_tc(
        g2, ea_p, iso_p, fobs_p, fmask_p,
        Wf1[:_H], Wf1[_H:2 * _H], Wf1[2 * _H:], bf1.reshape(1, _H),
        Wf2, bf2.reshape(1, 1),
        Wqa1[:_H], Wqa1[_H:2 * _H], Wqa1[2 * _H:], bqa1.reshape(1, _HH),
        Wqa2, bqa2.reshape(1, 1))

    return (pp[:, 0], flow[:_E, 0], h2, palog[:, 0], qlog[:_E, 0])


# speedup vs baseline: 1.3072x; 1.1990x over previous
import functools
import jax
import jax.numpy as jnp
from jax import lax
from jax.experimental import pallas as pl
from jax.experimental.pallas import tpu as pltpu
from jax.experimental.pallas import tpu_sc as plsc

_N = 50000
_E = 800000
_DIN = 128
_H = 64
_HH = 32
_DE = 8

_NC = 2
_NS = 16
_CH = 128
_EROWS = 6272
_EPAD = _EROWS * _CH
_RPT1 = _EROWS // _NS
_BLK1 = 56
_NB1 = _RPT1 // _BLK1
_ACC = 50048
_TACC = _ACC // _NS
_ZCH = 136
_DACC = 50176
_TDACC = _DACC // _NS
_DCH = 448
_DUMMY = _N
_RPT2 = _EROWS // (_NC * _NS)
_BLK2 = 28
_NB2 = _RPT2 // _BLK2
_GB1 = 2
_SLAB = 28
_NSLAB = _RPT1 // _SLAB
_SBLK = _SLAB // _GB1
_GB2 = 7
_NGB2 = _RPT2 // _GB2

_NBLK = 2000
_EBLK = 3136

_f32 = jnp.float32
_bf16 = jnp.bfloat16


def _sage_sc_body(with_deg, src_cat, dst_k1, h_tab, *refs):
    if with_deg:
        (agg_out, deg_out, isl_s, isl_d, big_a, big_b,
         gsem_a, gsem_b, ssem_a, ssem_b, ones_v, zdeg,
         acc_sh, deg_sh) = refs
    else:
        (agg_out, isl_s, isl_d, big_a, big_b,
         gsem_a, gsem_b, ssem_a, ssem_b, acc_sh) = refs
    cid = lax.axis_index("c")
    sid = lax.axis_index("s")

    z16 = jnp.zeros((16,), _f32)

    def _zrow(r, c):
        big_a[r, pl.ds(0, 16)] = z16
        big_a[r, pl.ds(16, 16)] = z16
        return c
    lax.fori_loop(0, _ZCH, _zrow, 0)

    base = sid * _TACC

    def _zacc(k, c):
        pltpu.sync_copy(big_a.at[pl.ds(0, _ZCH)],
                        acc_sh.at[pl.ds(base + k * _ZCH, _ZCH)])
        return c
    lax.fori_loop(0, _TACC // _ZCH, _zacc, 0)

    if with_deg:
        for k in range(8):
            ones_v[pl.ds(k * 16, 16)] = jnp.ones((16,), _f32)
        for k in range(8):
            zdeg[pl.ds(k * 16, 16)] = z16
        zdeg[pl.ds(_ZCH - 16, 16)] = z16

        def _zdg(k, c):
            pltpu.sync_copy(zdeg, deg_sh.at[pl.ds(base + k * _ZCH, _ZCH)])
            return c
        lax.fori_loop(0, _TACC // _ZCH, _zdg, 0)

    plsc.subcore_barrier()

    row0 = sid * _RPT1
    bufs = (big_a, big_b)
    gsems = (gsem_a, gsem_b)
    ssems = (ssem_a, ssem_b)

    def _fire(b, p):
        for j in range(_GB1):
            pltpu.async_copy(h_tab.at[isl_s.at[b * _GB1 + j]],
                             bufs[p].at[pl.ds(j * _CH, _CH)], gsems[p])

    def _drain_g(p):
        for j in range(_GB1):
            pltpu.make_async_copy(h_tab.at[isl_s.at[j]],
                                  bufs[p].at[pl.ds(j * _CH, _CH)],
                                  gsems[p]).wait()

    def _fire_s(b, p):
        for j in range(_GB1):
            pltpu.async_copy(bufs[p].at[pl.ds(j * _CH, _CH)],
                             acc_sh.at[isl_d.at[b * _GB1 + j]],
                             ssems[p], add=True)
        if with_deg:
            @pl.when(cid == 0)
            def _():
                for j in range(_GB1):
                    pltpu.sync_copy(ones_v, deg_sh.at[isl_d.at[b * _GB1 + j]],
                                    add=True)

    def _drain_s(p):
        for j in range(_GB1):
            pltpu.make_async_copy(bufs[p].at[pl.ds(j * _CH, _CH)],
                                  acc_sh.at[isl_d.at[j]],
                                  ssems[p]).wait()

    def _slab(si, c):
        r0 = row0 + si * _SLAB
        pltpu.sync_copy(src_cat.at[cid, pl.ds(r0, _SLAB)], isl_s)
        pltpu.sync_copy(dst_k1.at[pl.ds(r0, _SLAB)], isl_d)

        _fire(0, 0)
        _drain_g(0)
        _fire_s(0, 0)
        _fire(1, 1)

        def _steady(i, cc):
            b = 2 + 2 * i
            _drain_g(1)
            _fire_s(b - 1, 1)
            _drain_s(0)
            _fire(b, 0)
            _drain_g(0)
            _fire_s(b, 0)
            _drain_s(1)
            _fire(b + 1, 1)
            return cc
        lax.fori_loop(0, (_SBLK - 2) // 2, _steady, 0)

        _drain_g(1)
        _fire_s(_SBLK - 1, 1)
        _drain_s(0)
        _drain_s(1)
        return c
    lax.fori_loop(0, _NSLAB, _slab, 0)

    plsc.subcore_barrier()

    def _fl(k, c):
        off = base + k * _ZCH
        pltpu.sync_copy(acc_sh.at[pl.ds(off, _ZCH)], big_a.at[pl.ds(0, _ZCH)])
        pltpu.sync_copy(big_a.at[pl.ds(0, _ZCH)], agg_out.at[cid, pl.ds(off, _ZCH)])
        return c
    lax.fori_loop(0, _TACC // _ZCH, _fl, 0)

    if with_deg:
        @pl.when(cid == 0)
        def _():
            def _fd(k, c):
                off = base + k * _ZCH
                pltpu.sync_copy(deg_sh.at[pl.ds(off, _ZCH)], zdeg)
                pltpu.sync_copy(zdeg, deg_out.at[pl.ds(off, _ZCH)])
                return c
            lax.fori_loop(0, _TACC // _ZCH, _fd, 0)


def _sage_sc(src_cat, dst_k1, h_tab, with_deg):
    mesh = plsc.VectorSubcoreMesh(core_axis_name="c", subcore_axis_name="s")
    out_type = [jax.ShapeDtypeStruct((_NC, _ACC, _HH), _f32)]
    scratch = [
        pltpu.VMEM((_SLAB, _CH), jnp.int32),
        pltpu.VMEM((_SLAB, _CH), jnp.int32),
        pltpu.VMEM((_GB1 * _CH, _HH), _f32),
        pltpu.VMEM((_GB1 * _CH, _HH), _f32),
        pltpu.SemaphoreType.DMA,
        pltpu.SemaphoreType.DMA,
        pltpu.SemaphoreType.DMA,
        pltpu.SemaphoreType.DMA,
    ]
    if with_deg:
        out_type.append(jax.ShapeDtypeStruct((_ACC,), _f32))
        scratch += [pltpu.VMEM((_CH,), _f32), pltpu.VMEM((_ZCH,), _f32)]
    scratch.append(pltpu.VMEM_SHARED((_ACC, _HH), _f32))
    if with_deg:
        scratch.append(pltpu.VMEM_SHARED((_ACC,), _f32))
    return pl.kernel(
        functools.partial(_sage_sc_body, with_deg),
        out_type=out_type,
        mesh=mesh,
        scratch_types=scratch,
        compiler_params=pltpu.CompilerParams(use_tc_tiling_on_sc=False),
        name="sage_segment_sum_deg_sc" if with_deg else "sage_segment_sum_sc",
    )(src_cat, dst_k1, h_tab)


def _egather_sc_body(idx2, h_tab, out2, ix_a, ix_b, big_a, big_b,
                     gsem_a, gsem_b, wsem_a, wsem_b):
    cid = lax.axis_index("c")
    sid = lax.axis_index("s")
    wid = sid * _NC + cid
    row0 = wid * _RPT2
    bufs = (big_a, big_b)
    ixs = (ix_a, ix_b)
    gsems = (gsem_a, gsem_b)
    wsems = (wsem_a, wsem_b)
    nseg = _GB2 * _CH

    for g in range(2):
        def _fire(b, p):
            r0 = row0 + b * _GB2
            pltpu.sync_copy(idx2.at[g, pl.ds(r0, _GB2)], ixs[p])
            for j in range(_GB2):
                pltpu.async_copy(h_tab.at[ixs[p].at[j]],
                                 bufs[p].at[pl.ds(j * _CH, _CH)], gsems[p])

        def _drain_g(p):
            for j in range(_GB2):
                pltpu.make_async_copy(h_tab.at[ixs[p].at[j]],
                                      bufs[p].at[pl.ds(j * _CH, _CH)],
                                      gsems[p]).wait()

        def _fire_w(b, p):
            e0 = (row0 + b * _GB2) * _CH
            pltpu.async_copy(bufs[p], out2.at[g, pl.ds(e0, nseg)], wsems[p])

        def _drain_w(p):
            pltpu.make_async_copy(bufs[p], out2.at[g, pl.ds(0, nseg)],
                                  wsems[p]).wait()

        _fire(0, 0)
        _drain_g(0)
        _fire_w(0, 0)
        _fire(1, 1)

        def _steady(i, c):
            b = 2 + 2 * i
            _drain_g(1)
            _fire_w(b - 1, 1)
            _drain_w(0)
            _fire(b, 0)
            _drain_g(0)
            _fire_w(b, 0)
            _drain_w(1)
            _fire(b + 1, 1)
            return c
        lax.fori_loop(0, (_NGB2 - 2) // 2, _steady, 0)

        _drain_g(1)
        _fire_w(_NGB2 - 1, 1)
        _drain_w(0)
        _drain_w(1)


def _egather_sc(idx2, h_tab):
    mesh = plsc.VectorSubcoreMesh(core_axis_name="c", subcore_axis_name="s")
    return pl.kernel(
        _egather_sc_body,
        out_type=jax.ShapeDtypeStruct((2, _EPAD, _H), _f32),
        mesh=mesh,
        scratch_types=[
            pltpu.VMEM((_GB2, _CH), jnp.int32),
            pltpu.VMEM((_GB2, _CH), jnp.int32),
            pltpu.VMEM((_GB2 * _CH, _H), _f32),
            pltpu.VMEM((_GB2 * _CH, _H), _f32),
            pltpu.SemaphoreType.DMA,
            pltpu.SemaphoreType.DMA,
            pltpu.SemaphoreType.DMA,
            pltpu.SemaphoreType.DMA,
        ],
        compiler_params=pltpu.CompilerParams(use_tc_tiling_on_sc=False),
        name="edge_gather_sc",
    )(idx2, h_tab)


def _dot(a, b):
    return jnp.dot(a, b, preferred_element_type=_f32)


def _enc_body(x_ref, w_ref, b_ref, out_ref):
    h = jnp.maximum(_dot(x_ref[...], w_ref[...]) + b_ref[...], 0.0)
    out_ref[0] = h[:, :_HH]
    out_ref[1] = h[:, _HH:]


def _encoder(x, W, b):
    return pl.pallas_call(
        _enc_body,
        grid=(_N // _NBLK,),
        in_specs=[
            pl.BlockSpec((_NBLK, _DIN), lambda i: (i, 0)),
            pl.BlockSpec((_DIN, _H), lambda i: (0, 0)),
            pl.BlockSpec((1, _H), lambda i: (0, 0)),
        ],
        out_specs=pl.BlockSpec((2, _NBLK, _HH), lambda i: (0, i, 0)),
        out_shape=jax.ShapeDtypeStruct((2, _N, _HH), _f32),
    )(x, W, b)


def _sage_tc_body(h_ref, a_ref, d_ref, ws_ref, wn_ref, b_ref, out_ref):
    h = jnp.concatenate([h_ref[0], h_ref[1]], axis=1)
    agg = jnp.concatenate([a_ref[0], a_ref[1]], axis=1)
    r = 1.0 / jnp.maximum(d_ref[...], 1.0)
    h1 = jnp.maximum(
        _dot(h, ws_ref[...]) + _dot(agg * r, wn_ref[...]) + b_ref[...], 0.0)
    out_ref[0] = h1[:, :_HH]
    out_ref[1] = h1[:, _HH:]


def _sage_tc(h_st, agg_st, deg, Ws, Wn, b):
    return pl.pallas_call(
        _sage_tc_body,
        grid=(_N // _NBLK,),
        in_specs=[
            pl.BlockSpec((2, _NBLK, _HH), lambda i: (0, i, 0)),
            pl.BlockSpec((2, _NBLK, _HH), lambda i: (0, i, 0)),
            pl.BlockSpec((_NBLK, 1), lambda i: (i, 0)),
            pl.BlockSpec((_H, _H), lambda i: (0, 0)),
            pl.BlockSpec((_H, _H), lambda i: (0, 0)),
            pl.BlockSpec((1, _H), lambda i: (0, 0)),
        ],
        out_specs=pl.BlockSpec((2, _NBLK, _HH), lambda i: (0, i, 0)),
        out_shape=jax.ShapeDtypeStruct((2, _N, _HH), _f32),
    )(h_st, agg_st, deg, Ws, Wn, b)


def _node_final_body(h_ref, a_ref, d_ref, pobs_ref, pmask_ref,
                     ws_ref, wn_ref, b_ref,
                     wp1_ref, bp1_ref, wp2_ref, bp2_ref,
                     wpah_ref, wpap_ref, bpa1_ref, wpa2_ref, bpa2_ref,
                     hst_ref, hfull_ref, pp_ref, pa_ref):
    h = jnp.concatenate([h_ref[0], h_ref[1]], axis=1)
    agg = jnp.concatenate([a_ref[0], a_ref[1]], axis=1)
    r = 1.0 / jnp.maximum(d_ref[...], 1.0)
    h2 = jnp.maximum(
        _dot(h, ws_ref[...]) + _dot(agg * r, wn_ref[...]) + b_ref[...], 0.0)
    hst_ref[0] = h2[:, :_HH]
    hst_ref[1] = h2[:, _HH:]
    hfull_ref[...] = h2
    ph = jnp.maximum(_dot(h2, wp1_ref[...]) + bp1_ref[...], 0.0)
    pp = _dot(ph, wp2_ref[...]) + bp2_ref[...]
    pp_ref[...] = pp
    pobs = pobs_ref[...]
    pres = jnp.abs(pobs - pp)
    p4 = jnp.concatenate([pobs, pp, pres, pmask_ref[...]], axis=1)
    pa = jnp.maximum(
        _dot(h2, wpah_ref[...]) + _dot(p4, wpap_ref[...]) + bpa1_ref[...], 0.0)
    pa_ref[...] = _dot(pa, wpa2_ref[...]) + bpa2_ref[...]


def _node_final(h_st, agg_st, deg, pobs, pmask, Ws, Wn, b,
                Wp1, bp1, Wp2, bp2, Wpah, Wpap, bpa1, Wpa2, bpa2):
    full = lambda r, c: pl.BlockSpec((r, c), lambda i: (0, 0))
    return pl.pallas_call(
        _node_final_body,
        grid=(_N // _NBLK,),
        in_specs=[
            pl.BlockSpec((2, _NBLK, _HH), lambda i: (0, i, 0)),
            pl.BlockSpec((2, _NBLK, _HH), lambda i: (0, i, 0)),
            pl.BlockSpec((_NBLK, 1), lambda i: (i, 0)),
            pl.BlockSpec((_NBLK, 1), lambda i: (i, 0)),
            pl.BlockSpec((_NBLK, 1), lambda i: (i, 0)),
            full(_H, _H), full(_H, _H), full(1, _H),
            full(_H, _H), full(1, _H), full(_H, 1), full(1, 1),
            full(_H, _HH), full(4, _HH), full(1, _HH), full(_HH, 1),
            full(1, 1),
        ],
        out_specs=[
            pl.BlockSpec((2, _NBLK, _HH), lambda i: (0, i, 0)),
            pl.BlockSpec((_NBLK, _H), lambda i: (i, 0)),
            pl.BlockSpec((_NBLK, 1), lambda i: (i, 0)),
            pl.BlockSpec((_NBLK, 1), lambda i: (i, 0)),
        ],
        out_shape=[
            jax.ShapeDtypeStruct((2, _N, _HH), _f32),
            jax.ShapeDtypeStruct((_N, _H), _f32),
            jax.ShapeDtypeStruct((_N, 1), _f32),
            jax.ShapeDtypeStruct((_N, 1), _f32),
        ],
    )(h_st, agg_st, deg, pobs, pmask, Ws, Wn, b,
      Wp1, bp1, Wp2, bp2, Wpah, Wpap, bpa1, Wpa2, bpa2)


def _edge_body(g_ref, ea_ref, fobs_ref, fmask_ref,
               wfs_ref, wfd_ref, wfe_ref, bf1_ref, wf2_ref, bf2_ref,
               wqs_ref, wqd_ref, wqq_ref, bq1_ref, wq2_ref, bq2_ref,
               flow_ref, ql_ref):
    sp = g_ref[0]
    dp = g_ref[1]
    fh = jnp.maximum(
        _dot(sp, wfs_ref[...]) + _dot(dp, wfd_ref[...])
        + _dot(ea_ref[...], wfe_ref[...]) + bf1_ref[...], 0.0)
    flow = _dot(fh, wf2_ref[...]) + bf2_ref[...]
    flow_ref[...] = flow
    fobs = fobs_ref[...]
    qres = jnp.abs(fobs - flow)
    fmask = fmask_ref[...]
    q4 = jnp.concatenate(
        [fobs[:, 0:1], flow[:, 0:1], qres[:, 0:1], fmask[:, 0:1],
         fobs[:, 1:2], flow[:, 1:2], qres[:, 1:2], fmask[:, 1:2]], axis=1)
    qa = jnp.maximum(
        _dot(sp, wqs_ref[...]) + _dot(dp, wqd_ref[...])
        + _dot(q4, wqq_ref[...]) + bq1_ref[...], 0.0)
    ql_ref[...] = _dot(qa, wq2_ref[...]) + bq2_ref[...]


def _dup2(W):
    Z = jnp.zeros(W.shape, _f32)
    return jnp.concatenate(
        [jnp.concatenate([W, Z], axis=1), jnp.concatenate([Z, W], axis=1)],
        axis=0)


def _edge_tc(g2p, ea2, fobs2, fmask2,
             Wfs, Wfd, Wfe, bf1, Wf2, bf2,
             Wqs, Wqd, Wqq, bq1, Wq2, bq2):
    full = lambda r, c: pl.BlockSpec((r, c), lambda i: (0, 0))
    eb2 = _EBLK // 2
    return pl.pallas_call(
        _edge_body,
        grid=(_EPAD // _EBLK,),
        in_specs=[
            pl.BlockSpec((2, eb2, 128), lambda i: (0, i, 0)),
            pl.BlockSpec((eb2, 2 * _DE), lambda i: (i, 0)),
            pl.BlockSpec((eb2, 2), lambda i: (i, 0)),
            pl.BlockSpec((eb2, 2), lambda i: (i, 0)),
            full(128, 128), full(128, 128), full(2 * _DE, 128),
            full(1, 128), full(128, 2), full(1, 2),
            full(128, _H), full(128, _H), full(8, _H), full(1, _H),
            full(_H, 2), full(1, 2),
        ],
        out_specs=[
            pl.BlockSpec((eb2, 2), lambda i: (i, 0)),
            pl.BlockSpec((eb2, 2), lambda i: (i, 0)),
        ],
        out_shape=[
            jax.ShapeDtypeStruct((_EPAD // 2, 2), _f32),
            jax.ShapeDtypeStruct((_EPAD // 2, 2), _f32),
        ],
    )(g2p, ea2, fobs2, fmask2,
      _dup2(Wfs), _dup2(Wfd), _dup2(Wfe),
      jnp.concatenate([bf1, bf1], axis=1), _dup2(Wf2),
      jnp.concatenate([bf2, bf2], axis=1),
      _dup2(Wqs), _dup2(Wqd), _dup2(Wqq),
      jnp.concatenate([bq1, bq1], axis=1), _dup2(Wq2),
      jnp.concatenate([bq2, bq2], axis=1))


def kernel(x, edge_index, edge_attr, is_original_edge, pressure_obs, flow_obs,
           pressure_mask, flow_mask, W_enc, b_enc, Ws1, Wn1, b1, Ws2, Wn2, b2,
           Wp1, bp1, Wp2, bp2, Wf1, bf1, Wf2, bf2, Wpa1, bpa1, Wpa2, bpa2,
           Wqa1, bqa1, Wqa2, bqa2):
    src = edge_index[0]
    dst = edge_index[1]
    osrc = jnp.where(is_original_edge, src, 0)
    odst = jnp.where(is_original_edge, dst, 0)
    pad = _EPAD - _E
    zpad_i = jnp.zeros((pad,), jnp.int32)
    srcp = jnp.concatenate([src, zpad_i])
    dstp = jnp.concatenate([dst, jnp.full((pad,), _DUMMY, jnp.int32)])
    osrcp = jnp.concatenate([osrc, zpad_i])
    odstp = jnp.concatenate([odst, zpad_i])
    src_cat = jnp.stack([srcp, srcp + _N]).reshape(2, _EROWS, _CH)
    dst_k1 = dstp.reshape(_EROWS, _CH)
    idx2 = jnp.stack([osrcp, odstp]).reshape(2, _EROWS, _CH)

    h0_st = _encoder(x, W_enc, b_enc.reshape(1, _H))
    agg1_p, deg_p = _sage_sc(src_cat, dst_k1, h0_st.reshape(2 * _N, _HH), True)
    deg = deg_p.reshape(_ACC, 1)
    h1_st = _sage_tc(h0_st, agg1_p, deg, Ws1, Wn1, b1.reshape(1, _H))
    agg2_p, = _sage_sc(src_cat, dst_k1, h1_st.reshape(2 * _N, _HH), False)
    h2_st, h2, pp, palog = _node_final(
        h1_st, agg2_p, deg,
        pressure_obs.reshape(_N, 1), pressure_mask.reshape(_N, 1),
        Ws2, Wn2, b2.reshape(1, _H),
        Wp1, bp1.reshape(1, _H), Wp2, bp2.reshape(1, 1),
        Wpa1[:_H], Wpa1[_H:], bpa1.reshape(1, _HH), Wpa2,
        bpa2.reshape(1, 1))

    g2 = _egather_sc(idx2, h2)

    zpad_f = jnp.zeros((pad,), _f32)
    ea_m = jnp.where(is_original_edge[:, None], edge_attr, 0.0)
    ea2 = jnp.concatenate([ea_m, jnp.zeros((pad, _DE), _f32)]
                          ).reshape(_EPAD // 2, 2 * _DE)
    fobs2 = jnp.concatenate([flow_obs, zpad_f]).reshape(_EPAD // 2, 2)
    fmask2 = jnp.concatenate([flow_mask, zpad_f]).reshape(_EPAD // 2, 2)

    flow2, qlog2 = _edge_tc(
        g2.reshape(2, _EPAD // 2, 2 * _H), ea2, fobs2, fmask2,
        Wf1[:_H], Wf1[_H:2 * _H], Wf1[2 * _H:], bf1.reshape(1, _H),
        Wf2, bf2.reshape(1, 1),
        Wqa1[:_H], Wqa1[_H:2 * _H], Wqa1[2 * _H:], bqa1.reshape(1, _HH),
        Wqa2, bqa2.reshape(1, 1))

    return (pp[:, 0], flow2.reshape(_EPAD)[:_E], h2,
            palog[:, 0], qlog2.reshape(_EPAD)[:_E])


# speedup vs baseline: 1.4150x; 1.0825x over previous
import functools
import jax
import jax.numpy as jnp
from jax import lax
from jax.experimental import pallas as pl
from jax.experimental.pallas import tpu as pltpu
from jax.experimental.pallas import tpu_sc as plsc

_N = 50000
_E = 800000
_DIN = 128
_H = 64
_HH = 32
_DE = 8

_NC = 2
_NS = 16
_CH = 128
_EROWS = 6272
_EPAD = _EROWS * _CH
_RPT1 = _EROWS // _NS
_BLK1 = 56
_NB1 = _RPT1 // _BLK1
_ACC = 50048
_TACC = _ACC // _NS
_ZCH = 136
_DACC = 50176
_TDACC = _DACC // _NS
_DCH = 448
_DUMMY = _N
_RPT2 = _EROWS // (_NC * _NS)
_BLK2 = 28
_NB2 = _RPT2 // _BLK2
_GB1 = 2
_SLAB = 28
_NSLAB = _RPT1 // _SLAB
_SBLK = _SLAB // _GB1
_GB2 = 7
_NGB2 = _RPT2 // _GB2

_NBLK = 2000
_EBLK = 3136

_f32 = jnp.float32
_bf16 = jnp.bfloat16


def _sage_sc_body(with_deg, src_cat, dst_k1, h_tab, *refs):
    if with_deg:
        (agg_out, deg_out, isl_s, isl_d, big_a, big_b,
         gsem_a, gsem_b, ssem_a, ssem_b, dsem_a, dsem_b, ones_v, zdeg,
         acc_sh, deg_sh) = refs
        dsems = (dsem_a, dsem_b)
    else:
        (agg_out, isl_s, isl_d, big_a, big_b,
         gsem_a, gsem_b, ssem_a, ssem_b, acc_sh) = refs
    cid = lax.axis_index("c")
    sid = lax.axis_index("s")

    z16 = jnp.zeros((16,), _f32)

    def _zrow(r, c):
        big_a[r, pl.ds(0, 16)] = z16
        big_a[r, pl.ds(16, 16)] = z16
        return c
    lax.fori_loop(0, _ZCH, _zrow, 0)

    base = sid * _TACC

    def _zacc(k, c):
        pltpu.sync_copy(big_a.at[pl.ds(0, _ZCH)],
                        acc_sh.at[pl.ds(base + k * _ZCH, _ZCH)])
        return c
    lax.fori_loop(0, _TACC // _ZCH, _zacc, 0)

    if with_deg:
        for k in range(8):
            ones_v[pl.ds(k * 16, 16)] = jnp.ones((16,), _f32)
        for k in range(8):
            zdeg[pl.ds(k * 16, 16)] = z16
        zdeg[pl.ds(_ZCH - 16, 16)] = z16

        def _zdg(k, c):
            pltpu.sync_copy(zdeg, deg_sh.at[pl.ds(base + k * _ZCH, _ZCH)])
            return c
        lax.fori_loop(0, _TACC // _ZCH, _zdg, 0)

    plsc.subcore_barrier()

    row0 = sid * _RPT1
    bufs = (big_a, big_b)
    gsems = (gsem_a, gsem_b)
    ssems = (ssem_a, ssem_b)

    def _fire(b, p):
        for j in range(_GB1):
            pltpu.async_copy(h_tab.at[isl_s.at[b * _GB1 + j]],
                             bufs[p].at[pl.ds(j * _CH, _CH)], gsems[p])

    def _drain_g(p):
        for j in range(_GB1):
            pltpu.make_async_copy(h_tab.at[isl_s.at[j]],
                                  bufs[p].at[pl.ds(j * _CH, _CH)],
                                  gsems[p]).wait()

    def _fire_s(b, p):
        for j in range(_GB1):
            pltpu.async_copy(bufs[p].at[pl.ds(j * _CH, _CH)],
                             acc_sh.at[isl_d.at[b * _GB1 + j]],
                             ssems[p], add=True)
        if with_deg:
            @pl.when(cid == 0)
            def _():
                for j in range(_GB1):
                    pltpu.async_copy(ones_v, deg_sh.at[isl_d.at[b * _GB1 + j]],
                                     dsems[p], add=True)

    def _drain_s(p):
        for j in range(_GB1):
            pltpu.make_async_copy(bufs[p].at[pl.ds(j * _CH, _CH)],
                                  acc_sh.at[isl_d.at[j]],
                                  ssems[p]).wait()
        if with_deg:
            @pl.when(cid == 0)
            def _():
                for j in range(_GB1):
                    pltpu.make_async_copy(ones_v, deg_sh.at[isl_d.at[j]],
                                          dsems[p]).wait()

    def _slab(si, c):
        r0 = row0 + si * _SLAB
        pltpu.sync_copy(src_cat.at[cid, pl.ds(r0, _SLAB)], isl_s)
        pltpu.sync_copy(dst_k1.at[pl.ds(r0, _SLAB)], isl_d)

        _fire(0, 0)
        _drain_g(0)
        _fire_s(0, 0)
        _fire(1, 1)

        def _steady(i, cc):
            b = 2 + 2 * i
            _drain_g(1)
            _fire_s(b - 1, 1)
            _drain_s(0)
            _fire(b, 0)
            _drain_g(0)
            _fire_s(b, 0)
            _drain_s(1)
            _fire(b + 1, 1)
            return cc
        lax.fori_loop(0, (_SBLK - 2) // 2, _steady, 0)

        _drain_g(1)
        _fire_s(_SBLK - 1, 1)
        _drain_s(0)
        _drain_s(1)
        return c
    lax.fori_loop(0, _NSLAB, _slab, 0)

    plsc.subcore_barrier()

    def _fl(k, c):
        off = base + k * _ZCH
        pltpu.sync_copy(acc_sh.at[pl.ds(off, _ZCH)], big_a.at[pl.ds(0, _ZCH)])
        pltpu.sync_copy(big_a.at[pl.ds(0, _ZCH)], agg_out.at[cid, pl.ds(off, _ZCH)])
        return c
    lax.fori_loop(0, _TACC // _ZCH, _fl, 0)

    if with_deg:
        @pl.when(cid == 0)
        def _():
            def _fd(k, c):
                off = base + k * _ZCH
                pltpu.sync_copy(deg_sh.at[pl.ds(off, _ZCH)], zdeg)
                pltpu.sync_copy(zdeg, deg_out.at[pl.ds(off, _ZCH)])
                return c
            lax.fori_loop(0, _TACC // _ZCH, _fd, 0)


def _sage_sc(src_cat, dst_k1, h_tab, with_deg):
    mesh = plsc.VectorSubcoreMesh(core_axis_name="c", subcore_axis_name="s")
    out_type = [jax.ShapeDtypeStruct((_NC, _ACC, _HH), _f32)]
    scratch = [
        pltpu.VMEM((_SLAB, _CH), jnp.int32),
        pltpu.VMEM((_SLAB, _CH), jnp.int32),
        pltpu.VMEM((_GB1 * _CH, _HH), _f32),
        pltpu.VMEM((_GB1 * _CH, _HH), _f32),
        pltpu.SemaphoreType.DMA,
        pltpu.SemaphoreType.DMA,
        pltpu.SemaphoreType.DMA,
        pltpu.SemaphoreType.DMA,
    ]
    if with_deg:
        out_type.append(jax.ShapeDtypeStruct((_ACC,), _f32))
        scratch += [pltpu.SemaphoreType.DMA, pltpu.SemaphoreType.DMA,
                    pltpu.VMEM((_CH,), _f32), pltpu.VMEM((_ZCH,), _f32)]
    scratch.append(pltpu.VMEM_SHARED((_ACC, _HH), _f32))
    if with_deg:
        scratch.append(pltpu.VMEM_SHARED((_ACC,), _f32))
    return pl.kernel(
        functools.partial(_sage_sc_body, with_deg),
        out_type=out_type,
        mesh=mesh,
        scratch_types=scratch,
        compiler_params=pltpu.CompilerParams(use_tc_tiling_on_sc=False),
        name="sage_segment_sum_deg_sc" if with_deg else "sage_segment_sum_sc",
    )(src_cat, dst_k1, h_tab)


def _egather_sc_body(idx2, h_tab, out2, ix_a, ix_b, big_a, big_b,
                     gsem_a, gsem_b, wsem_a, wsem_b):
    cid = lax.axis_index("c")
    sid = lax.axis_index("s")
    wid = sid * _NC + cid
    row0 = wid * _RPT2
    bufs = (big_a, big_b)
    ixs = (ix_a, ix_b)
    gsems = (gsem_a, gsem_b)
    wsems = (wsem_a, wsem_b)
    nseg = _GB2 * _CH

    for g in range(2):
        def _fire(b, p):
            r0 = row0 + b * _GB2
            pltpu.sync_copy(idx2.at[g, pl.ds(r0, _GB2)], ixs[p])
            for j in range(_GB2):
                pltpu.async_copy(h_tab.at[ixs[p].at[j]],
                                 bufs[p].at[pl.ds(j * _CH, _CH)], gsems[p])

        def _drain_g(p):
            for j in range(_GB2):
                pltpu.make_async_copy(h_tab.at[ixs[p].at[j]],
                                      bufs[p].at[pl.ds(j * _CH, _CH)],
                                      gsems[p]).wait()

        def _fire_w(b, p):
            e0 = (row0 + b * _GB2) * _CH
            pltpu.async_copy(bufs[p], out2.at[g, pl.ds(e0, nseg)], wsems[p])

        def _drain_w(p):
            pltpu.make_async_copy(bufs[p], out2.at[g, pl.ds(0, nseg)],
                                  wsems[p]).wait()

        _fire(0, 0)
        _drain_g(0)
        _fire_w(0, 0)
        _fire(1, 1)

        def _steady(i, c):
            b = 2 + 2 * i
            _drain_g(1)
            _fire_w(b - 1, 1)
            _drain_w(0)
            _fire(b, 0)
            _drain_g(0)
            _fire_w(b, 0)
            _drain_w(1)
            _fire(b + 1, 1)
            return c
        lax.fori_loop(0, (_NGB2 - 2) // 2, _steady, 0)

        _drain_g(1)
        _fire_w(_NGB2 - 1, 1)
        _drain_w(0)
        _drain_w(1)


def _egather_sc(idx2, h_tab):
    mesh = plsc.VectorSubcoreMesh(core_axis_name="c", subcore_axis_name="s")
    return pl.kernel(
        _egather_sc_body,
        out_type=jax.ShapeDtypeStruct((2, _EPAD, _H), _f32),
        mesh=mesh,
        scratch_types=[
            pltpu.VMEM((_GB2, _CH), jnp.int32),
            pltpu.VMEM((_GB2, _CH), jnp.int32),
            pltpu.VMEM((_GB2 * _CH, _H), _f32),
            pltpu.VMEM((_GB2 * _CH, _H), _f32),
            pltpu.SemaphoreType.DMA,
            pltpu.SemaphoreType.DMA,
            pltpu.SemaphoreType.DMA,
            pltpu.SemaphoreType.DMA,
        ],
        compiler_params=pltpu.CompilerParams(use_tc_tiling_on_sc=False),
        name="edge_gather_sc",
    )(idx2, h_tab)


def _dot(a, b):
    return jnp.dot(a, b, preferred_element_type=_f32)


_IBLK = 224


def _idx_body(src_ref, dst_ref, iso_ref, cat_ref, dk1_ref, idx2_ref):
    i = pl.program_id(0)
    row0 = i * _IBLK
    gid = jax.lax.broadcasted_iota(jnp.int32, (_IBLK, _CH), 0) + row0
    valid = gid < (_E // _CH)
    src = src_ref[...]
    dst = dst_ref[...]
    iso = iso_ref[...]
    s0 = jnp.where(valid, src, 0)
    cat_ref[0] = s0
    cat_ref[1] = s0 + _N
    dk1_ref[...] = jnp.where(valid, dst, _DUMMY)
    orig = jnp.logical_and(valid, iso > 0)
    idx2_ref[0] = jnp.where(orig, src, 0)
    idx2_ref[1] = jnp.where(orig, dst, 0)


def _idx_prep(src2d, dst2d, iso2d):
    return pl.pallas_call(
        _idx_body,
        grid=(_EROWS // _IBLK,),
        in_specs=[
            pl.BlockSpec((_IBLK, _CH), lambda i: (i, 0)),
            pl.BlockSpec((_IBLK, _CH), lambda i: (i, 0)),
            pl.BlockSpec((_IBLK, _CH), lambda i: (i, 0)),
        ],
        out_specs=[
            pl.BlockSpec((2, _IBLK, _CH), lambda i: (0, i, 0)),
            pl.BlockSpec((_IBLK, _CH), lambda i: (i, 0)),
            pl.BlockSpec((2, _IBLK, _CH), lambda i: (0, i, 0)),
        ],
        out_shape=[
            jax.ShapeDtypeStruct((2, _EROWS, _CH), jnp.int32),
            jax.ShapeDtypeStruct((_EROWS, _CH), jnp.int32),
            jax.ShapeDtypeStruct((2, _EROWS, _CH), jnp.int32),
        ],
    )(src2d, dst2d, iso2d)


def _enc_body(x_ref, w_ref, b_ref, out_ref):
    h = jnp.maximum(_dot(x_ref[...], w_ref[...]) + b_ref[...], 0.0)
    out_ref[0] = h[:, :_HH]
    out_ref[1] = h[:, _HH:]


def _encoder(x, W, b):
    return pl.pallas_call(
        _enc_body,
        grid=(_N // _NBLK,),
        in_specs=[
            pl.BlockSpec((_NBLK, _DIN), lambda i: (i, 0)),
            pl.BlockSpec((_DIN, _H), lambda i: (0, 0)),
            pl.BlockSpec((1, _H), lambda i: (0, 0)),
        ],
        out_specs=pl.BlockSpec((2, _NBLK, _HH), lambda i: (0, i, 0)),
        out_shape=jax.ShapeDtypeStruct((2, _N, _HH), _f32),
    )(x, W, b)


def _sage_tc_body(h_ref, a_ref, d_ref, ws_ref, wn_ref, b_ref, out_ref):
    h = jnp.concatenate([h_ref[0], h_ref[1]], axis=1)
    agg = jnp.concatenate([a_ref[0], a_ref[1]], axis=1)
    r = 1.0 / jnp.maximum(d_ref[...], 1.0)
    h1 = jnp.maximum(
        _dot(h, ws_ref[...]) + _dot(agg * r, wn_ref[...]) + b_ref[...], 0.0)
    out_ref[0] = h1[:, :_HH]
    out_ref[1] = h1[:, _HH:]


def _sage_tc(h_st, agg_st, deg, Ws, Wn, b):
    return pl.pallas_call(
        _sage_tc_body,
        grid=(_N // _NBLK,),
        in_specs=[
            pl.BlockSpec((2, _NBLK, _HH), lambda i: (0, i, 0)),
            pl.BlockSpec((2, _NBLK, _HH), lambda i: (0, i, 0)),
            pl.BlockSpec((_NBLK, 1), lambda i: (i, 0)),
            pl.BlockSpec((_H, _H), lambda i: (0, 0)),
            pl.BlockSpec((_H, _H), lambda i: (0, 0)),
            pl.BlockSpec((1, _H), lambda i: (0, 0)),
        ],
        out_specs=pl.BlockSpec((2, _NBLK, _HH), lambda i: (0, i, 0)),
        out_shape=jax.ShapeDtypeStruct((2, _N, _HH), _f32),
    )(h_st, agg_st, deg, Ws, Wn, b)


def _node_final_body(h_ref, a_ref, d_ref, pobs_ref, pmask_ref,
                     ws_ref, wn_ref, b_ref,
                     wp1_ref, bp1_ref, wp2_ref, bp2_ref,
                     wpah_ref, wpap_ref, bpa1_ref, wpa2_ref, bpa2_ref,
                     hst_ref, hfull_ref, pp_ref, pa_ref):
    h = jnp.concatenate([h_ref[0], h_ref[1]], axis=1)
    agg = jnp.concatenate([a_ref[0], a_ref[1]], axis=1)
    r = 1.0 / jnp.maximum(d_ref[...], 1.0)
    h2 = jnp.maximum(
        _dot(h, ws_ref[...]) + _dot(agg * r, wn_ref[...]) + b_ref[...], 0.0)
    hst_ref[0] = h2[:, :_HH]
    hst_ref[1] = h2[:, _HH:]
    hfull_ref[...] = h2
    ph = jnp.maximum(_dot(h2, wp1_ref[...]) + bp1_ref[...], 0.0)
    pp = _dot(ph, wp2_ref[...]) + bp2_ref[...]
    pp_ref[...] = pp
    pobs = pobs_ref[...]
    pres = jnp.abs(pobs - pp)
    p4 = jnp.concatenate([pobs, pp, pres, pmask_ref[...]], axis=1)
    pa = jnp.maximum(
        _dot(h2, wpah_ref[...]) + _dot(p4, wpap_ref[...]) + bpa1_ref[...], 0.0)
    pa_ref[...] = _dot(pa, wpa2_ref[...]) + bpa2_ref[...]


def _node_final(h_st, agg_st, deg, pobs, pmask, Ws, Wn, b,
                Wp1, bp1, Wp2, bp2, Wpah, Wpap, bpa1, Wpa2, bpa2):
    full = lambda r, c: pl.BlockSpec((r, c), lambda i: (0, 0))
    return pl.pallas_call(
        _node_final_body,
        grid=(_N // _NBLK,),
        in_specs=[
            pl.BlockSpec((2, _NBLK, _HH), lambda i: (0, i, 0)),
            pl.BlockSpec((2, _NBLK, _HH), lambda i: (0, i, 0)),
            pl.BlockSpec((_NBLK, 1), lambda i: (i, 0)),
            pl.BlockSpec((_NBLK, 1), lambda i: (i, 0)),
            pl.BlockSpec((_NBLK, 1), lambda i: (i, 0)),
            full(_H, _H), full(_H, _H), full(1, _H),
            full(_H, _H), full(1, _H), full(_H, 1), full(1, 1),
            full(_H, _HH), full(4, _HH), full(1, _HH), full(_HH, 1),
            full(1, 1),
        ],
        out_specs=[
            pl.BlockSpec((2, _NBLK, _HH), lambda i: (0, i, 0)),
            pl.BlockSpec((_NBLK, _H), lambda i: (i, 0)),
            pl.BlockSpec((_NBLK, 1), lambda i: (i, 0)),
            pl.BlockSpec((_NBLK, 1), lambda i: (i, 0)),
        ],
        out_shape=[
            jax.ShapeDtypeStruct((2, _N, _HH), _f32),
            jax.ShapeDtypeStruct((_N, _H), _f32),
            jax.ShapeDtypeStruct((_N, 1), _f32),
            jax.ShapeDtypeStruct((_N, 1), _f32),
        ],
    )(h_st, agg_st, deg, pobs, pmask, Ws, Wn, b,
      Wp1, bp1, Wp2, bp2, Wpah, Wpap, bpa1, Wpa2, bpa2)


def _edge_body(g_ref, ea_ref, fobs_ref, fmask_ref,
               wfs_ref, wfd_ref, wfe_ref, bf1_ref, wf2_ref, bf2_ref,
               wqs_ref, wqd_ref, wqq_ref, bq1_ref, wq2_ref, bq2_ref,
               flow_ref, ql_ref):
    sp = g_ref[0]
    dp = g_ref[1]
    fh = jnp.maximum(
        _dot(sp, wfs_ref[...]) + _dot(dp, wfd_ref[...])
        + _dot(ea_ref[...], wfe_ref[...]) + bf1_ref[...], 0.0)
    flow = _dot(fh, wf2_ref[...]) + bf2_ref[...]
    flow_ref[...] = flow
    fobs = fobs_ref[...]
    qres = jnp.abs(fobs - flow)
    fmask = fmask_ref[...]
    q4 = jnp.concatenate(
        [fobs[:, 0:1], flow[:, 0:1], qres[:, 0:1], fmask[:, 0:1],
         fobs[:, 1:2], flow[:, 1:2], qres[:, 1:2], fmask[:, 1:2]], axis=1)
    qa = jnp.maximum(
        _dot(sp, wqs_ref[...]) + _dot(dp, wqd_ref[...])
        + _dot(q4, wqq_ref[...]) + bq1_ref[...], 0.0)
    ql_ref[...] = _dot(qa, wq2_ref[...]) + bq2_ref[...]


def _dup2(W):
    Z = jnp.zeros(W.shape, _f32)
    return jnp.concatenate(
        [jnp.concatenate([W, Z], axis=1), jnp.concatenate([Z, W], axis=1)],
        axis=0)


def _edge_tc(g2p, ea2, fobs2, fmask2,
             Wfs, Wfd, Wfe, bf1, Wf2, bf2,
             Wqs, Wqd, Wqq, bq1, Wq2, bq2):
    full = lambda r, c: pl.BlockSpec((r, c), lambda i: (0, 0))
    eb2 = _EBLK // 2
    return pl.pallas_call(
        _edge_body,
        grid=(_EPAD // _EBLK,),
        in_specs=[
            pl.BlockSpec((2, eb2, 128), lambda i: (0, i, 0)),
            pl.BlockSpec((eb2, 2 * _DE), lambda i: (i, 0)),
            pl.BlockSpec((eb2, 2), lambda i: (i, 0)),
            pl.BlockSpec((eb2, 2), lambda i: (i, 0)),
            full(128, 128), full(128, 128), full(2 * _DE, 128),
            full(1, 128), full(128, 2), full(1, 2),
            full(128, _H), full(128, _H), full(8, _H), full(1, _H),
            full(_H, 2), full(1, 2),
        ],
        out_specs=[
            pl.BlockSpec((eb2, 2), lambda i: (i, 0)),
            pl.BlockSpec((eb2, 2), lambda i: (i, 0)),
        ],
        out_shape=[
            jax.ShapeDtypeStruct((_EPAD // 2, 2), _f32),
            jax.ShapeDtypeStruct((_EPAD // 2, 2), _f32),
        ],
    )(g2p, ea2, fobs2, fmask2,
      _dup2(Wfs), _dup2(Wfd), _dup2(Wfe),
      jnp.concatenate([bf1, bf1], axis=1), _dup2(Wf2),
      jnp.concatenate([bf2, bf2], axis=1),
      _dup2(Wqs), _dup2(Wqd), _dup2(Wqq),
      jnp.concatenate([bq1, bq1], axis=1), _dup2(Wq2),
      jnp.concatenate([bq2, bq2], axis=1))


def kernel(x, edge_index, edge_attr, is_original_edge, pressure_obs, flow_obs,
           pressure_mask, flow_mask, W_enc, b_enc, Ws1, Wn1, b1, Ws2, Wn2, b2,
           Wp1, bp1, Wp2, bp2, Wf1, bf1, Wf2, bf2, Wpa1, bpa1, Wpa2, bpa2,
           Wqa1, bqa1, Wqa2, bqa2):
    src2d = edge_index[0].reshape(_E // _CH, _CH)
    dst2d = edge_index[1].reshape(_E // _CH, _CH)
    iso2d = is_original_edge.astype(jnp.int32).reshape(_E // _CH, _CH)
    src_cat, dst_k1, idx2 = _idx_prep(src2d, dst2d, iso2d)

    h0_st = _encoder(x, W_enc, b_enc.reshape(1, _H))
    agg1_p, deg_p = _sage_sc(src_cat, dst_k1, h0_st.reshape(2 * _N, _HH), True)
    deg = deg_p.reshape(_ACC, 1)
    h1_st = _sage_tc(h0_st, agg1_p, deg, Ws1, Wn1, b1.reshape(1, _H))
    agg2_p, = _sage_sc(src_cat, dst_k1, h1_st.reshape(2 * _N, _HH), False)
    h2_st, h2, pp, palog = _node_final(
        h1_st, agg2_p, deg,
        pressure_obs.reshape(_N, 1), pressure_mask.reshape(_N, 1),
        Ws2, Wn2, b2.reshape(1, _H),
        Wp1, bp1.reshape(1, _H), Wp2, bp2.reshape(1, 1),
        Wpa1[:_H], Wpa1[_H:], bpa1.reshape(1, _HH), Wpa2,
        bpa2.reshape(1, 1))

    g2 = _egather_sc(idx2, h2)

    ea2 = jnp.where(is_original_edge[:, None], edge_attr, 0.0
                    ).reshape(_E // 2, 2 * _DE)
    fobs2 = flow_obs.reshape(_E // 2, 2)
    fmask2 = flow_mask.reshape(_E // 2, 2)

    flow2, qlog2 = _edge_tc(
        g2.reshape(2, _EPAD // 2, 2 * _H), ea2, fobs2, fmask2,
        Wf1[:_H], Wf1[_H:2 * _H], Wf1[2 * _H:], bf1.reshape(1, _H),
        Wf2, bf2.reshape(1, 1),
        Wqa1[:_H], Wqa1[_H:2 * _H], Wqa1[2 * _H:], bqa1.reshape(1, _HH),
        Wqa2, bqa2.reshape(1, 1))

    return (pp[:, 0], flow2.reshape(_EPAD)[:_E], h2,
            palog[:, 0], qlog2.reshape(_EPAD)[:_E])
